# baseline (speedup 1.0000x reference)
import functools

import jax
import jax.numpy as jnp
from jax import lax
from jax.experimental import pallas as pl
from jax.experimental.pallas import tpu as pltpu
from jax.experimental.pallas import tpu_sc as plsc

N = 10000
NPAD = 10240
E = 320000
B = 64
HID = 128
NCOV = 8
NC, NS, L = 2, 16, 16
NW = NC * NS
EPW = E // NW
CH = 128
EPWP = 10240
CPW = EPWP // CH
STRIPE = NPAD // NS
RB = 1280
GR = NPAD // RB


def _sc_mesh():
    return plsc.VectorSubcoreMesh(core_axis_name="c", subcore_axis_name="s",
                                  num_cores=NC, num_subcores=NS)


@functools.cache
def _deg_kernel():
    return functools.partial(
        pl.kernel,
        out_type=jax.ShapeDtypeStruct((NW, NPAD), jnp.float32),
        mesh=_sc_mesh(),
        compiler_params=pltpu.CompilerParams(needs_layout_passes=False),
        scratch_types=[
            pltpu.VMEM((EPW,), jnp.int32),
            pltpu.VMEM((NPAD,), jnp.float32),
        ],
    )(_deg_body)


def _deg_call(col):
    return _deg_kernel()(col)


def _deg_body(col_hbm, out_hbm, colv, acc):
    cid = lax.axis_index("c")
    sid = lax.axis_index("s")
    wid = cid * NS + sid
    pltpu.sync_copy(col_hbm.at[pl.ds(wid * EPW, EPW)], colv)
    zeros = jnp.zeros((L,), jnp.float32)
    ones = jnp.ones((L,), jnp.float32)

    def zbody(i, _):
        acc[pl.ds(i * L, L)] = zeros
        return _

    lax.fori_loop(0, NPAD // L, zbody, None)

    def hbody(i, _):
        idx = colv[pl.ds(i * L, L)]
        plsc.addupdate_scatter(acc, [idx], ones)
        return _

    lax.fori_loop(0, EPW // L, hbody, None)
    pltpu.sync_copy(acc, out_hbm.at[wid])


@functools.cache
def _scatter_kernel():
    return functools.partial(
        pl.kernel,
        out_type=jax.ShapeDtypeStruct((NC * 3 * NPAD, HID), jnp.float32),
        mesh=_sc_mesh(),
        compiler_params=pltpu.CompilerParams(needs_layout_passes=False),
        scratch_types=[
            pltpu.VMEM((CPW, CH), jnp.int32),
            pltpu.VMEM((CPW, CH), jnp.int32),
            pltpu.VMEM((CH, HID), jnp.float32),
            pltpu.VMEM_SHARED((NPAD, HID), jnp.float32),
            pltpu.SemaphoreType.DMA,
        ],
    )(_scatter_body)


def _scatter_call(row3, col, z):
    return _scatter_kernel()(row3, col, z)


def _scatter_body(row3_hbm, col_hbm, z_hbm, out_hbm,
                  rowv, colv, gbuf, acc_sh, gsem):
    cid = lax.axis_index("c")
    sid = lax.axis_index("s")
    wid = cid * NS + sid
    zeros = jnp.zeros((L,), jnp.float32)
    pltpu.sync_copy(col_hbm.at[wid], colv)

    for s in range(3):
        def zb(t, _):
            gbuf[t // 8, pl.ds((t % 8) * L, L)] = zeros
            return _

        lax.fori_loop(0, CH * HID // L, zb, None)
        for k in range(STRIPE // CH):
            pltpu.sync_copy(gbuf,
                            acc_sh.at[pl.ds(sid * STRIPE + k * CH, CH)])
        pltpu.sync_copy(row3_hbm.at[s * NW + wid], rowv)
        plsc.subcore_barrier()

        def chunk(i, _):
            pltpu.async_copy(z_hbm.at[rowv.at[i]], gbuf, gsem).wait()
            pltpu.sync_copy(gbuf, acc_sh.at[colv.at[i]], add=True)
            return _

        lax.fori_loop(0, CPW, chunk, None)
        plsc.subcore_barrier()
        obase = (cid * 3 + s) * NPAD + sid * STRIPE
        pltpu.sync_copy(acc_sh.at[pl.ds(sid * STRIPE, STRIPE)],
                        out_hbm.at[pl.ds(obase, STRIPE)])


def _nt(a, b):
    return lax.dot_general(a, b, (((1,), (1,)), ((), ())),
                           preferred_element_type=jnp.float32)


def _ka_body(x_ref, degp_ref, lin_w_ref, lin_b_ref, c1_w_ref,
             h_ref, z1_ref, dinv_ref):
    deg = jnp.sum(degp_ref[...], axis=0, keepdims=True) + 2.0
    dlane = lax.rsqrt(deg)
    ones = jnp.ones((1, HID), jnp.float32)
    dinv = lax.dot_general(dlane, ones, (((0,), (0,)), ((), ())),
                           preferred_element_type=jnp.float32)
    dinv_ref[...] = dinv
    for s in range(3):
        xs = x_ref[:, s * HID:(s + 1) * HID]
        hs = _nt(xs, lin_w_ref[...]) + lin_b_ref[...]
        xw = _nt(hs, c1_w_ref[...])
        h_ref[s] = hs
        z1_ref[s] = dinv * xw


def _kb_body(dinv_ref, h_ref, z1_ref, p_ref, c1_b_ref, c2_w_ref,
             out1_ref, z2_ref):
    dinv = dinv_ref[...]
    agg = p_ref[0, 0] + p_ref[1, 0]
    conv1 = dinv * agg + 2.0 * dinv * z1_ref[0] + c1_b_ref[...]
    o1 = jax.nn.relu(conv1) + h_ref[0]
    out1_ref[0] = o1
    z2_ref[0] = dinv * _nt(o1, c2_w_ref[...])


def _kc_body(dinv_ref, out1_ref, z2_ref, p_ref, c2_b_ref, batch_ref,
             seq_ref, pooled, cnt):
    i = pl.program_id(1)
    dinv = dinv_ref[...]
    agg = p_ref[0, 0] + p_ref[1, 0]
    o2 = dinv * agg + 2.0 * dinv * z2_ref[0] + c2_b_ref[...] + out1_ref[0]
    bt = batch_ref[...]
    ohT = (jnp.broadcast_to(bt, (B, RB))
           == lax.broadcasted_iota(jnp.int32, (B, RB), 0)).astype(jnp.float32)

    @pl.when(i == 0)
    def _():
        pooled[...] = jnp.zeros_like(pooled)
        cnt[...] = jnp.zeros_like(cnt)

    pooled[...] += lax.dot_general(ohT, o2, (((1,), (0,)), ((), ())),
                                   preferred_element_type=jnp.float32)
    cnt[...] += jnp.sum(ohT, axis=1, keepdims=True)

    @pl.when(i == GR - 1)
    def _():
        seq_ref[0] = pooled[...] / jnp.maximum(cnt[...], 1.0)


def _kd_body(seq_ref, cov_ref, w_ih_ref, w_hh_ref, b_ih_ref, b_hh_ref,
             w0_w_ref, w0_b_ref, aw1_ref, aw2_ref, ab_ref,
             l1_w_ref, l1_b_ref, l2a_ref, l2b_ref, l2_b_ref,
             h0_ref, c0_ref, attn_ref, out_ref):
    h = h0_ref[...]
    c = c0_ref[...]
    hs = []
    for t in range(3):
        xt = seq_ref[t]
        g = (_nt(xt, w_ih_ref[...]) + b_ih_ref[...]
             + _nt(h, w_hh_ref[...]) + b_hh_ref[...])
        ii = jax.nn.sigmoid(g[:, 0 * HID:1 * HID])
        ff = jax.nn.sigmoid(g[:, 1 * HID:2 * HID])
        gg = jnp.tanh(g[:, 2 * HID:3 * HID])
        oo = jax.nn.sigmoid(g[:, 3 * HID:4 * HID])
        c = ff * c + ii * gg
        h = oo * jnp.tanh(c)
        hs.append(h)
    hT = hs[-1]
    aw1 = aw1_ref[...]
    aw2 = aw2_ref[...]
    ab = ab_ref[0, 0]
    rs, ws = [], []
    for t in range(3):
        rt = jnp.tanh(_nt(hs[t], w0_w_ref[...]) + w0_b_ref[...])
        wt = (jnp.sum(rt * aw1, axis=1, keepdims=True)
              + jnp.sum(hT * aw2, axis=1, keepdims=True) + ab)
        rs.append(rt)
        ws.append(wt)
    m = jnp.maximum(ws[0], jnp.maximum(ws[1], ws[2]))
    es = [jnp.exp(w - m) for w in ws]
    tot = es[0] + es[1] + es[2]
    als = [e / tot for e in es]
    feat = als[0] * rs[0] + als[1] * rs[1] + als[2] * rs[2]
    l1o = jax.nn.relu(_nt(feat, l1_w_ref[...]) + l1_b_ref[...])
    out = (_nt(l1o, l2a_ref[...]) + _nt(cov_ref[...], l2b_ref[...])
           + l2_b_ref[...])
    attn_ref[...] = jnp.concatenate(
        [als[0], als[1], als[2], jnp.zeros((B, HID - 3), jnp.float32)], axis=1)
    out_ref[...] = jnp.concatenate(
        [out, jnp.zeros((B, HID - 2), jnp.float32)], axis=1)


def _full(shape):
    return pl.BlockSpec(shape, lambda *_: tuple(0 for _ in shape))


_ka = pl.pallas_call(
    _ka_body,
    grid=(GR,),
    in_specs=[
        pl.BlockSpec((RB, 3 * HID), lambda i: (i, 0)),
        pl.BlockSpec((NW, RB), lambda i: (0, i)),
        _full((HID, HID)),
        _full((1, HID)),
        _full((HID, HID)),
    ],
    out_specs=[
        pl.BlockSpec((3, RB, HID), lambda i: (0, i, 0)),
        pl.BlockSpec((3, RB, HID), lambda i: (0, i, 0)),
        pl.BlockSpec((RB, HID), lambda i: (i, 0)),
    ],
    out_shape=[
        jax.ShapeDtypeStruct((3, NPAD, HID), jnp.float32),
        jax.ShapeDtypeStruct((3, NPAD, HID), jnp.float32),
        jax.ShapeDtypeStruct((NPAD, HID), jnp.float32),
    ],
)

_kb = pl.pallas_call(
    _kb_body,
    grid=(3, GR),
    in_specs=[
        pl.BlockSpec((RB, HID), lambda s, i: (i, 0)),
        pl.BlockSpec((1, RB, HID), lambda s, i: (s, i, 0)),
        pl.BlockSpec((1, RB, HID), lambda s, i: (s, i, 0)),
        pl.BlockSpec((NC, 1, RB, HID), lambda s, i: (0, s, i, 0)),
        _full((1, HID)),
        _full((HID, HID)),
    ],
    out_specs=[
        pl.BlockSpec((1, RB, HID), lambda s, i: (s, i, 0)),
        pl.BlockSpec((1, RB, HID), lambda s, i: (s, i, 0)),
    ],
    out_shape=[
        jax.ShapeDtypeStruct((3, NPAD, HID), jnp.float32),
        jax.ShapeDtypeStruct((3, NPAD, HID), jnp.float32),
    ],
)

_kc = pl.pallas_call(
    _kc_body,
    grid=(3, GR),
    in_specs=[
        pl.BlockSpec((RB, HID), lambda s, i: (i, 0)),
        pl.BlockSpec((1, RB, HID), lambda s, i: (s, i, 0)),
        pl.BlockSpec((1, RB, HID), lambda s, i: (s, i, 0)),
        pl.BlockSpec((NC, 1, RB, HID), lambda s, i: (0, s, i, 0)),
        _full((1, HID)),
        pl.BlockSpec((1, RB), lambda s, i: (0, i)),
    ],
    out_specs=[pl.BlockSpec((1, B, HID), lambda s, i: (s, 0, 0))],
    out_shape=[jax.ShapeDtypeStruct((3, B, HID), jnp.float32)],
    scratch_shapes=[
        pltpu.VMEM((B, HID), jnp.float32),
        pltpu.VMEM((B, 1), jnp.float32),
    ],
)

_kd = pl.pallas_call(
    _kd_body,
    out_shape=[
        jax.ShapeDtypeStruct((B, HID), jnp.float32),
        jax.ShapeDtypeStruct((B, HID), jnp.float32),
    ],
)


def kernel(x, edge_index, cov, batch, lin_w, lin_b, c1_w, c1_b, c2_w, c2_b,
           w_ih, w_hh, b_ih, b_hh, w0_w, w0_b, attn_w, attn_b,
           l1_w, l1_b, l2_w, l2_b, h0, c0):
    f32 = jnp.float32
    x_pad = jnp.pad(x, ((0, NPAD - N), (0, 0)))
    batch_pad = jnp.pad(batch, (0, NPAD - N),
                        constant_values=B).reshape(1, NPAD)
    row = edge_index[0]
    col = edge_index[1]
    row_w = jnp.pad(row.reshape(NW, EPW), ((0, 0), (0, EPWP - EPW)))
    col_w = jnp.pad(col.reshape(NW, EPW), ((0, 0), (0, EPWP - EPW)),
                    constant_values=NPAD - 1)
    row3 = (row_w[None] + (jnp.arange(3, dtype=jnp.int32)
                           * NPAD)[:, None, None])
    row3 = row3.reshape(3 * NW, CPW, CH)
    col3 = col_w.reshape(NW, CPW, CH)

    degp = _deg_call(col)
    enc_h, z1, dinv_b = _ka(x_pad, degp, lin_w, lin_b.reshape(1, HID), c1_w)
    p1 = _scatter_call(row3, col3, z1.reshape(3 * NPAD, HID))
    p1 = p1.reshape(NC, 3, NPAD, HID)
    out1, z2 = _kb(dinv_b, enc_h, z1, p1, c1_b.reshape(1, HID), c2_w)
    p2 = _scatter_call(row3, col3, z2.reshape(3 * NPAD, HID))
    p2 = p2.reshape(NC, 3, NPAD, HID)
    (seq,) = _kc(dinv_b, out1, z2, p2, c2_b.reshape(1, HID), batch_pad)

    attn_p, out_p = _kd(
        seq, cov.astype(f32), w_ih, w_hh,
        b_ih.reshape(1, 4 * HID), b_hh.reshape(1, 4 * HID),
        w0_w, w0_b.reshape(1, HID),
        attn_w[:, :HID], attn_w[:, HID:], attn_b.reshape(1, 1),
        l1_w, l1_b.reshape(1, 8),
        l2_w[:, :8], l2_w[:, 8:], l2_b.reshape(1, 2),
        h0[0], c0[0])
    return (attn_p[:, :3], out_p[:, :2])


# speedup vs baseline: 1.0985x; 1.0985x over previous
import functools

import jax
import jax.numpy as jnp
from jax import lax
from jax.experimental import pallas as pl
from jax.experimental.pallas import tpu as pltpu
from jax.experimental.pallas import tpu_sc as plsc

N = 10000
NPAD = 10240
E = 320000
B = 64
HID = 128
NCOV = 8
NC, NS, L = 2, 16, 16
NW = NC * NS
EPW = E // NW
CH = 128
EPWP = 10240
CPW = EPWP // CH
STRIPE = NPAD // NS
RB = 1280
GR = NPAD // RB


def _sc_mesh():
    return plsc.VectorSubcoreMesh(core_axis_name="c", subcore_axis_name="s",
                                  num_cores=NC, num_subcores=NS)


@functools.cache
def _deg_kernel():
    return functools.partial(
        pl.kernel,
        out_type=jax.ShapeDtypeStruct((NW, NPAD), jnp.float32),
        mesh=_sc_mesh(),
        compiler_params=pltpu.CompilerParams(needs_layout_passes=False),
        scratch_types=[
            pltpu.VMEM((EPW,), jnp.int32),
            pltpu.VMEM((NPAD,), jnp.float32),
        ],
    )(_deg_body)


def _deg_call(col):
    return _deg_kernel()(col)


def _deg_body(col_hbm, out_hbm, colv, acc):
    cid = lax.axis_index("c")
    sid = lax.axis_index("s")
    wid = cid * NS + sid
    pltpu.sync_copy(col_hbm.at[pl.ds(wid * EPW, EPW)], colv)
    zeros = jnp.zeros((L,), jnp.float32)
    ones = jnp.ones((L,), jnp.float32)

    def zbody(i, _):
        acc[pl.ds(i * L, L)] = zeros
        return _

    lax.fori_loop(0, NPAD // L, zbody, None)

    def hbody(i, _):
        idx = colv[pl.ds(i * L, L)]
        plsc.addupdate_scatter(acc, [idx], ones)
        return _

    lax.fori_loop(0, EPW // L, hbody, None)
    pltpu.sync_copy(acc, out_hbm.at[wid])


@functools.cache
def _scatter_kernel():
    return functools.partial(
        pl.kernel,
        out_type=jax.ShapeDtypeStruct((NC * 3 * NPAD, HID), jnp.float32),
        mesh=_sc_mesh(),
        compiler_params=pltpu.CompilerParams(needs_layout_passes=False),
        scratch_types=[
            pltpu.VMEM((2, CH), jnp.int32),
            pltpu.VMEM((CPW, CH), jnp.int32),
            pltpu.VMEM((2, CH, HID), jnp.float32),
            pltpu.VMEM_SHARED((NPAD, HID), jnp.float32),
            pltpu.SemaphoreType.DMA((2,)),
            pltpu.SemaphoreType.DMA((2,)),
        ],
    )(_scatter_body)


def _scatter_call(row3, col, z):
    return _scatter_kernel()(row3, col, z)


def _scatter_body(row3_hbm, col_hbm, z_hbm, out_hbm,
                  rowi, colv, gbuf, acc_sh, gsem, isem):
    cid = lax.axis_index("c")
    sid = lax.axis_index("s")
    wid = cid * NS + sid
    zeros = jnp.zeros((L,), jnp.float32)
    pltpu.sync_copy(col_hbm.at[wid], colv)

    for s in range(3):
        def zb(t, _):
            gbuf[0, t // 8, pl.ds((t % 8) * L, L)] = zeros
            return _

        lax.fori_loop(0, CH * HID // L, zb, None)
        for k in range(STRIPE // CH):
            pltpu.sync_copy(gbuf.at[0],
                            acc_sh.at[pl.ds(sid * STRIPE + k * CH, CH)])
        rbase = (s * NW + wid) * CPW
        pltpu.sync_copy(row3_hbm.at[rbase], rowi.at[0])
        pltpu.async_copy(row3_hbm.at[rbase + 1], rowi.at[1], isem.at[1])
        plsc.subcore_barrier()
        pltpu.async_copy(z_hbm.at[rowi.at[0]], gbuf.at[0], gsem.at[0])

        def chunk(i, _):
            b = lax.rem(i, 2)
            pltpu.make_async_copy(
                z_hbm.at[rowi.at[b]], gbuf.at[b], gsem.at[b]).wait()

            @pl.when(i < CPW - 1)
            def _nxt():
                pltpu.make_async_copy(
                    row3_hbm.at[rbase], rowi.at[1 - b], isem.at[1 - b]).wait()
                pltpu.async_copy(z_hbm.at[rowi.at[1 - b]], gbuf.at[1 - b],
                                 gsem.at[1 - b])

            @pl.when(i < CPW - 2)
            def _pref():
                pltpu.async_copy(row3_hbm.at[rbase + i + 2], rowi.at[b],
                                 isem.at[b])

            pltpu.sync_copy(gbuf.at[b], acc_sh.at[colv.at[i]], add=True)
            return _

        lax.fori_loop(0, CPW, chunk, None)
        plsc.subcore_barrier()
        obase = (cid * 3 + s) * NPAD + sid * STRIPE
        pltpu.sync_copy(acc_sh.at[pl.ds(sid * STRIPE, STRIPE)],
                        out_hbm.at[pl.ds(obase, STRIPE)])


def _nt(a, b):
    return lax.dot_general(a, b, (((1,), (1,)), ((), ())),
                           preferred_element_type=jnp.float32)


def _ka_body(x_ref, degp_ref, lin_w_ref, lin_b_ref, c1_w_ref,
             h_ref, z1_ref, dinv_ref):
    deg = jnp.sum(degp_ref[...], axis=0, keepdims=True) + 2.0
    dlane = lax.rsqrt(deg)
    ones = jnp.ones((1, HID), jnp.float32)
    dinv = lax.dot_general(dlane, ones, (((0,), (0,)), ((), ())),
                           preferred_element_type=jnp.float32)
    dinv_ref[...] = dinv
    for s in range(3):
        xs = x_ref[:, s * HID:(s + 1) * HID]
        hs = _nt(xs, lin_w_ref[...]) + lin_b_ref[...]
        xw = _nt(hs, c1_w_ref[...])
        h_ref[s] = hs
        z1_ref[s] = dinv * xw


def _kb_body(dinv_ref, h_ref, z1_ref, p_ref, c1_b_ref, c2_w_ref,
             out1_ref, z2_ref):
    dinv = dinv_ref[...]
    agg = p_ref[0, 0] + p_ref[1, 0]
    conv1 = dinv * agg + 2.0 * dinv * z1_ref[0] + c1_b_ref[...]
    o1 = jax.nn.relu(conv1) + h_ref[0]
    out1_ref[0] = o1
    z2_ref[0] = dinv * _nt(o1, c2_w_ref[...])


def _kc_body(dinv_ref, out1_ref, z2_ref, p_ref, c2_b_ref, batch_ref,
             seq_ref, pooled, cnt):
    i = pl.program_id(1)
    dinv = dinv_ref[...]
    agg = p_ref[0, 0] + p_ref[1, 0]
    o2 = dinv * agg + 2.0 * dinv * z2_ref[0] + c2_b_ref[...] + out1_ref[0]
    bt = batch_ref[...]
    ohT = (jnp.broadcast_to(bt, (B, RB))
           == lax.broadcasted_iota(jnp.int32, (B, RB), 0)).astype(jnp.float32)

    @pl.when(i == 0)
    def _():
        pooled[...] = jnp.zeros_like(pooled)
        cnt[...] = jnp.zeros_like(cnt)

    pooled[...] += lax.dot_general(ohT, o2, (((1,), (0,)), ((), ())),
                                   preferred_element_type=jnp.float32)
    cnt[...] += jnp.sum(ohT, axis=1, keepdims=True)

    @pl.when(i == GR - 1)
    def _():
        seq_ref[0] = pooled[...] / jnp.maximum(cnt[...], 1.0)


def _kd_body(seq_ref, cov_ref, w_ih_ref, w_hh_ref, b_ih_ref, b_hh_ref,
             w0_w_ref, w0_b_ref, aw1_ref, aw2_ref, ab_ref,
             l1_w_ref, l1_b_ref, l2a_ref, l2b_ref, l2_b_ref,
             h0_ref, c0_ref, attn_ref, out_ref):
    h = h0_ref[...]
    c = c0_ref[...]
    hs = []
    for t in range(3):
        xt = seq_ref[t]
        g = (_nt(xt, w_ih_ref[...]) + b_ih_ref[...]
             + _nt(h, w_hh_ref[...]) + b_hh_ref[...])
        ii = jax.nn.sigmoid(g[:, 0 * HID:1 * HID])
        ff = jax.nn.sigmoid(g[:, 1 * HID:2 * HID])
        gg = jnp.tanh(g[:, 2 * HID:3 * HID])
        oo = jax.nn.sigmoid(g[:, 3 * HID:4 * HID])
        c = ff * c + ii * gg
        h = oo * jnp.tanh(c)
        hs.append(h)
    hT = hs[-1]
    aw1 = aw1_ref[...]
    aw2 = aw2_ref[...]
    ab = ab_ref[0, 0]
    rs, ws = [], []
    for t in range(3):
        rt = jnp.tanh(_nt(hs[t], w0_w_ref[...]) + w0_b_ref[...])
        wt = (jnp.sum(rt * aw1, axis=1, keepdims=True)
              + jnp.sum(hT * aw2, axis=1, keepdims=True) + ab)
        rs.append(rt)
        ws.append(wt)
    m = jnp.maximum(ws[0], jnp.maximum(ws[1], ws[2]))
    es = [jnp.exp(w - m) for w in ws]
    tot = es[0] + es[1] + es[2]
    als = [e / tot for e in es]
    feat = als[0] * rs[0] + als[1] * rs[1] + als[2] * rs[2]
    l1o = jax.nn.relu(_nt(feat, l1_w_ref[...]) + l1_b_ref[...])
    out = (_nt(l1o, l2a_ref[...]) + _nt(cov_ref[...], l2b_ref[...])
           + l2_b_ref[...])
    attn_ref[...] = jnp.concatenate(
        [als[0], als[1], als[2], jnp.zeros((B, HID - 3), jnp.float32)], axis=1)
    out_ref[...] = jnp.concatenate(
        [out, jnp.zeros((B, HID - 2), jnp.float32)], axis=1)


def _full(shape):
    return pl.BlockSpec(shape, lambda *_: tuple(0 for _ in shape))


_ka = pl.pallas_call(
    _ka_body,
    grid=(GR,),
    in_specs=[
        pl.BlockSpec((RB, 3 * HID), lambda i: (i, 0)),
        pl.BlockSpec((NW, RB), lambda i: (0, i)),
        _full((HID, HID)),
        _full((1, HID)),
        _full((HID, HID)),
    ],
    out_specs=[
        pl.BlockSpec((3, RB, HID), lambda i: (0, i, 0)),
        pl.BlockSpec((3, RB, HID), lambda i: (0, i, 0)),
        pl.BlockSpec((RB, HID), lambda i: (i, 0)),
    ],
    out_shape=[
        jax.ShapeDtypeStruct((3, NPAD, HID), jnp.float32),
        jax.ShapeDtypeStruct((3, NPAD, HID), jnp.float32),
        jax.ShapeDtypeStruct((NPAD, HID), jnp.float32),
    ],
)

_kb = pl.pallas_call(
    _kb_body,
    grid=(3, GR),
    in_specs=[
        pl.BlockSpec((RB, HID), lambda s, i: (i, 0)),
        pl.BlockSpec((1, RB, HID), lambda s, i: (s, i, 0)),
        pl.BlockSpec((1, RB, HID), lambda s, i: (s, i, 0)),
        pl.BlockSpec((NC, 1, RB, HID), lambda s, i: (0, s, i, 0)),
        _full((1, HID)),
        _full((HID, HID)),
    ],
    out_specs=[
        pl.BlockSpec((1, RB, HID), lambda s, i: (s, i, 0)),
        pl.BlockSpec((1, RB, HID), lambda s, i: (s, i, 0)),
    ],
    out_shape=[
        jax.ShapeDtypeStruct((3, NPAD, HID), jnp.float32),
        jax.ShapeDtypeStruct((3, NPAD, HID), jnp.float32),
    ],
)

_kc = pl.pallas_call(
    _kc_body,
    grid=(3, GR),
    in_specs=[
        pl.BlockSpec((RB, HID), lambda s, i: (i, 0)),
        pl.BlockSpec((1, RB, HID), lambda s, i: (s, i, 0)),
        pl.BlockSpec((1, RB, HID), lambda s, i: (s, i, 0)),
        pl.BlockSpec((NC, 1, RB, HID), lambda s, i: (0, s, i, 0)),
        _full((1, HID)),
        pl.BlockSpec((1, RB), lambda s, i: (0, i)),
    ],
    out_specs=[pl.BlockSpec((1, B, HID), lambda s, i: (s, 0, 0))],
    out_shape=[jax.ShapeDtypeStruct((3, B, HID), jnp.float32)],
    scratch_shapes=[
        pltpu.VMEM((B, HID), jnp.float32),
        pltpu.VMEM((B, 1), jnp.float32),
    ],
)

_kd = pl.pallas_call(
    _kd_body,
    out_shape=[
        jax.ShapeDtypeStruct((B, HID), jnp.float32),
        jax.ShapeDtypeStruct((B, HID), jnp.float32),
    ],
)


def kernel(x, edge_index, cov, batch, lin_w, lin_b, c1_w, c1_b, c2_w, c2_b,
           w_ih, w_hh, b_ih, b_hh, w0_w, w0_b, attn_w, attn_b,
           l1_w, l1_b, l2_w, l2_b, h0, c0):
    f32 = jnp.float32
    x_pad = jnp.pad(x, ((0, NPAD - N), (0, 0)))
    batch_pad = jnp.pad(batch, (0, NPAD - N),
                        constant_values=B).reshape(1, NPAD)
    row = edge_index[0]
    col = edge_index[1]
    row_w = jnp.pad(row.reshape(NW, EPW), ((0, 0), (0, EPWP - EPW)))
    col_w = jnp.pad(col.reshape(NW, EPW), ((0, 0), (0, EPWP - EPW)),
                    constant_values=NPAD - 1)
    row3 = (row_w[None] + (jnp.arange(3, dtype=jnp.int32)
                           * NPAD)[:, None, None])
    row3 = row3.reshape(3 * NW * CPW, CH)
    col3 = col_w.reshape(NW, CPW, CH)

    degp = _deg_call(col)
    enc_h, z1, dinv_b = _ka(x_pad, degp, lin_w, lin_b.reshape(1, HID), c1_w)
    p1 = _scatter_call(row3, col3, z1.reshape(3 * NPAD, HID))
    p1 = p1.reshape(NC, 3, NPAD, HID)
    out1, z2 = _kb(dinv_b, enc_h, z1, p1, c1_b.reshape(1, HID), c2_w)
    p2 = _scatter_call(row3, col3, z2.reshape(3 * NPAD, HID))
    p2 = p2.reshape(NC, 3, NPAD, HID)
    (seq,) = _kc(dinv_b, out1, z2, p2, c2_b.reshape(1, HID), batch_pad)

    attn_p, out_p = _kd(
        seq, cov.astype(f32), w_ih, w_hh,
        b_ih.reshape(1, 4 * HID), b_hh.reshape(1, 4 * HID),
        w0_w, w0_b.reshape(1, HID),
        attn_w[:, :HID], attn_w[:, HID:], attn_b.reshape(1, 1),
        l1_w, l1_b.reshape(1, 8),
        l2_w[:, :8], l2_w[:, 8:], l2_b.reshape(1, 2),
        h0[0], c0[0])
    return (attn_p[:, :3], out_p[:, :2])


# speedup vs baseline: 2.3527x; 2.1417x over previous
import functools

import jax
import jax.numpy as jnp
from jax import lax
from jax.experimental import pallas as pl
from jax.experimental.pallas import tpu as pltpu
from jax.experimental.pallas import tpu_sc as plsc

N = 10000
NPAD = 10240
E = 320000
B = 64
HID = 128
NCOV = 8
NC, NS, L = 2, 16, 16
NW = NC * NS
EPW = E // NW
CH = 80
EPWP = EPW
CPW = EPWP // CH
CPAIR = 62
STRIPE = NPAD // NS
RB = 1280
GR = NPAD // RB


def _sc_mesh():
    return plsc.VectorSubcoreMesh(core_axis_name="c", subcore_axis_name="s",
                                  num_cores=NC, num_subcores=NS)


@functools.cache
def _deg_kernel():
    return functools.partial(
        pl.kernel,
        out_type=jax.ShapeDtypeStruct((NW, NPAD), jnp.float32),
        mesh=_sc_mesh(),
        compiler_params=pltpu.CompilerParams(needs_layout_passes=False),
        scratch_types=[
            pltpu.VMEM((EPW,), jnp.int32),
            pltpu.VMEM((NPAD,), jnp.float32),
        ],
    )(_deg_body)


def _deg_call(col):
    return _deg_kernel()(col)


def _deg_body(col_hbm, out_hbm, colv, acc):
    cid = lax.axis_index("c")
    sid = lax.axis_index("s")
    wid = cid * NS + sid
    pltpu.sync_copy(col_hbm.at[pl.ds(wid * EPW, EPW)], colv)
    zeros = jnp.zeros((L,), jnp.float32)
    ones = jnp.ones((L,), jnp.float32)

    def zbody(i, _):
        acc[pl.ds(i * L, L)] = zeros
        return _

    lax.fori_loop(0, NPAD // L, zbody, None)

    def hbody(i, _):
        idx = colv[pl.ds(i * L, L)]
        plsc.addupdate_scatter(acc, [idx], ones)
        return _

    lax.fori_loop(0, EPW // L, hbody, None)
    pltpu.sync_copy(acc, out_hbm.at[wid])


@functools.cache
def _scatter_kernel():
    return functools.partial(
        pl.kernel,
        out_type=jax.ShapeDtypeStruct((NC * 3 * NPAD, HID), jnp.float32),
        mesh=_sc_mesh(),
        compiler_params=pltpu.CompilerParams(needs_layout_passes=False),
        scratch_types=[
            pltpu.VMEM((CH,), jnp.int32),
            pltpu.VMEM((CH,), jnp.int32),
            pltpu.VMEM((CH,), jnp.int32),
            pltpu.VMEM((CH, HID), jnp.float32),
            pltpu.VMEM((CH, HID), jnp.float32),
            pltpu.VMEM_SHARED((NPAD, HID), jnp.float32),
            pltpu.SemaphoreType.DMA,
            pltpu.SemaphoreType.DMA,
        ],
    )(_scatter_body)


def _scatter_call(row3, col, z):
    return _scatter_kernel()(row3, col, z)


def _scatter_body(row3_hbm, col_hbm, z_hbm, out_hbm,
                  rowa, rowb, colv, gbufa, gbufb, acc_sh, gsema, gsemb):
    cid = lax.axis_index("c")
    sid = lax.axis_index("s")
    wid = cid * NS + sid
    ebase = wid * EPW
    zeros = jnp.zeros((L,), jnp.float32)

    for s in range(3):
        def zb(t, _):
            gbufa[t // 8, pl.ds((t % 8) * L, L)] = zeros
            return _

        lax.fori_loop(0, CH * HID // L, zb, None)
        for k in range(STRIPE // CH):
            pltpu.sync_copy(gbufa,
                            acc_sh.at[pl.ds(sid * STRIPE + k * CH, CH)])
        plsc.subcore_barrier()

        sbase = s * E + ebase
        pltpu.sync_copy(row3_hbm.at[pl.ds(sbase, CH)], rowa)
        pltpu.async_copy(z_hbm.at[rowa], gbufa, gsema)

        def pair(g, _):
            e0 = g * 2
            pltpu.sync_copy(row3_hbm.at[pl.ds(sbase + (e0 + 1) * CH, CH)],
                            rowb)
            pltpu.async_copy(z_hbm.at[rowb], gbufb, gsemb)
            pltpu.make_async_copy(z_hbm.at[rowa], gbufa, gsema).wait()
            pltpu.sync_copy(col_hbm.at[pl.ds(ebase + e0 * CH, CH)], colv)
            pltpu.sync_copy(gbufa, acc_sh.at[colv], add=True)
            pltpu.sync_copy(row3_hbm.at[pl.ds(sbase + (e0 + 2) * CH, CH)],
                            rowa)
            pltpu.async_copy(z_hbm.at[rowa], gbufa, gsema)
            pltpu.make_async_copy(z_hbm.at[rowb], gbufb, gsemb).wait()
            pltpu.sync_copy(col_hbm.at[pl.ds(ebase + (e0 + 1) * CH, CH)],
                            colv)
            pltpu.sync_copy(gbufb, acc_sh.at[colv], add=True)
            return _

        lax.fori_loop(0, CPAIR, pair, None)
        pltpu.make_async_copy(z_hbm.at[rowa], gbufa, gsema).wait()
        pltpu.sync_copy(col_hbm.at[pl.ds(ebase + (CPW - 1) * CH, CH)], colv)
        pltpu.sync_copy(gbufa, acc_sh.at[colv], add=True)

        plsc.subcore_barrier()
        obase = (cid * 3 + s) * NPAD + sid * STRIPE
        pltpu.sync_copy(acc_sh.at[pl.ds(sid * STRIPE, STRIPE)],
                        out_hbm.at[pl.ds(obase, STRIPE)])


def _nt(a, b):
    return lax.dot_general(a, b, (((1,), (1,)), ((), ())),
                           preferred_element_type=jnp.float32)


def _ka_body(x_ref, degp_ref, lin_w_ref, lin_b_ref, c1_w_ref,
             h_ref, z1_ref, dinv_ref):
    deg = jnp.sum(degp_ref[...], axis=0, keepdims=True) + 2.0
    dlane = lax.rsqrt(deg)
    ones = jnp.ones((1, HID), jnp.float32)
    dinv = lax.dot_general(dlane, ones, (((0,), (0,)), ((), ())),
                           preferred_element_type=jnp.float32)
    dinv_ref[...] = dinv
    for s in range(3):
        xs = x_ref[:, s * HID:(s + 1) * HID]
        hs = _nt(xs, lin_w_ref[...]) + lin_b_ref[...]
        xw = _nt(hs, c1_w_ref[...])
        h_ref[s] = hs
        z1_ref[s] = dinv * xw


def _kb_body(dinv_ref, h_ref, z1_ref, p_ref, c1_b_ref, c2_w_ref,
             out1_ref, z2_ref):
    dinv = dinv_ref[...]
    agg = p_ref[0, 0] + p_ref[1, 0]
    conv1 = dinv * agg + 2.0 * dinv * z1_ref[0] + c1_b_ref[...]
    o1 = jax.nn.relu(conv1) + h_ref[0]
    out1_ref[0] = o1
    z2_ref[0] = dinv * _nt(o1, c2_w_ref[...])


def _kc_body(dinv_ref, out1_ref, z2_ref, p_ref, c2_b_ref, batch_ref,
             seq_ref, pooled, cnt):
    i = pl.program_id(1)
    dinv = dinv_ref[...]
    agg = p_ref[0, 0] + p_ref[1, 0]
    o2 = dinv * agg + 2.0 * dinv * z2_ref[0] + c2_b_ref[...] + out1_ref[0]
    bt = batch_ref[...]
    ohT = (jnp.broadcast_to(bt, (B, RB))
           == lax.broadcasted_iota(jnp.int32, (B, RB), 0)).astype(jnp.float32)

    @pl.when(i == 0)
    def _():
        pooled[...] = jnp.zeros_like(pooled)
        cnt[...] = jnp.zeros_like(cnt)

    pooled[...] += lax.dot_general(ohT, o2, (((1,), (0,)), ((), ())),
                                   preferred_element_type=jnp.float32)
    cnt[...] += jnp.sum(ohT, axis=1, keepdims=True)

    @pl.when(i == GR - 1)
    def _():
        seq_ref[0] = pooled[...] / jnp.maximum(cnt[...], 1.0)


def _kd_body(seq_ref, cov_ref, w_ih_ref, w_hh_ref, b_ih_ref, b_hh_ref,
             w0_w_ref, w0_b_ref, aw1_ref, aw2_ref, ab_ref,
             l1_w_ref, l1_b_ref, l2a_ref, l2b_ref, l2_b_ref,
             h0_ref, c0_ref, attn_ref, out_ref):
    h = h0_ref[...]
    c = c0_ref[...]
    hs = []
    for t in range(3):
        xt = seq_ref[t]
        g = (_nt(xt, w_ih_ref[...]) + b_ih_ref[...]
             + _nt(h, w_hh_ref[...]) + b_hh_ref[...])
        ii = jax.nn.sigmoid(g[:, 0 * HID:1 * HID])
        ff = jax.nn.sigmoid(g[:, 1 * HID:2 * HID])
        gg = jnp.tanh(g[:, 2 * HID:3 * HID])
        oo = jax.nn.sigmoid(g[:, 3 * HID:4 * HID])
        c = ff * c + ii * gg
        h = oo * jnp.tanh(c)
        hs.append(h)
    hT = hs[-1]
    aw1 = aw1_ref[...]
    aw2 = aw2_ref[...]
    ab = ab_ref[0, 0]
    rs, ws = [], []
    for t in range(3):
        rt = jnp.tanh(_nt(hs[t], w0_w_ref[...]) + w0_b_ref[...])
        wt = (jnp.sum(rt * aw1, axis=1, keepdims=True)
              + jnp.sum(hT * aw2, axis=1, keepdims=True) + ab)
        rs.append(rt)
        ws.append(wt)
    m = jnp.maximum(ws[0], jnp.maximum(ws[1], ws[2]))
    es = [jnp.exp(w - m) for w in ws]
    tot = es[0] + es[1] + es[2]
    als = [e / tot for e in es]
    feat = als[0] * rs[0] + als[1] * rs[1] + als[2] * rs[2]
    l1o = jax.nn.relu(_nt(feat, l1_w_ref[...]) + l1_b_ref[...])
    out = (_nt(l1o, l2a_ref[...]) + _nt(cov_ref[...], l2b_ref[...])
           + l2_b_ref[...])
    attn_ref[...] = jnp.concatenate(
        [als[0], als[1], als[2], jnp.zeros((B, HID - 3), jnp.float32)], axis=1)
    out_ref[...] = jnp.concatenate(
        [out, jnp.zeros((B, HID - 2), jnp.float32)], axis=1)


def _full(shape):
    return pl.BlockSpec(shape, lambda *_: tuple(0 for _ in shape))


_ka = pl.pallas_call(
    _ka_body,
    grid=(GR,),
    in_specs=[
        pl.BlockSpec((RB, 3 * HID), lambda i: (i, 0)),
        pl.BlockSpec((NW, RB), lambda i: (0, i)),
        _full((HID, HID)),
        _full((1, HID)),
        _full((HID, HID)),
    ],
    out_specs=[
        pl.BlockSpec((3, RB, HID), lambda i: (0, i, 0)),
        pl.BlockSpec((3, RB, HID), lambda i: (0, i, 0)),
        pl.BlockSpec((RB, HID), lambda i: (i, 0)),
    ],
    out_shape=[
        jax.ShapeDtypeStruct((3, NPAD, HID), jnp.float32),
        jax.ShapeDtypeStruct((3, NPAD, HID), jnp.float32),
        jax.ShapeDtypeStruct((NPAD, HID), jnp.float32),
    ],
)

_kb = pl.pallas_call(
    _kb_body,
    grid=(3, GR),
    in_specs=[
        pl.BlockSpec((RB, HID), lambda s, i: (i, 0)),
        pl.BlockSpec((1, RB, HID), lambda s, i: (s, i, 0)),
        pl.BlockSpec((1, RB, HID), lambda s, i: (s, i, 0)),
        pl.BlockSpec((NC, 1, RB, HID), lambda s, i: (0, s, i, 0)),
        _full((1, HID)),
        _full((HID, HID)),
    ],
    out_specs=[
        pl.BlockSpec((1, RB, HID), lambda s, i: (s, i, 0)),
        pl.BlockSpec((1, RB, HID), lambda s, i: (s, i, 0)),
    ],
    out_shape=[
        jax.ShapeDtypeStruct((3, NPAD, HID), jnp.float32),
        jax.ShapeDtypeStruct((3, NPAD, HID), jnp.float32),
    ],
)

_kc = pl.pallas_call(
    _kc_body,
    grid=(3, GR),
    in_specs=[
        pl.BlockSpec((RB, HID), lambda s, i: (i, 0)),
        pl.BlockSpec((1, RB, HID), lambda s, i: (s, i, 0)),
        pl.BlockSpec((1, RB, HID), lambda s, i: (s, i, 0)),
        pl.BlockSpec((NC, 1, RB, HID), lambda s, i: (0, s, i, 0)),
        _full((1, HID)),
        pl.BlockSpec((1, RB), lambda s, i: (0, i)),
    ],
    out_specs=[pl.BlockSpec((1, B, HID), lambda s, i: (s, 0, 0))],
    out_shape=[jax.ShapeDtypeStruct((3, B, HID), jnp.float32)],
    scratch_shapes=[
        pltpu.VMEM((B, HID), jnp.float32),
        pltpu.VMEM((B, 1), jnp.float32),
    ],
)

_kd = pl.pallas_call(
    _kd_body,
    out_shape=[
        jax.ShapeDtypeStruct((B, HID), jnp.float32),
        jax.ShapeDtypeStruct((B, HID), jnp.float32),
    ],
)


def kernel(x, edge_index, cov, batch, lin_w, lin_b, c1_w, c1_b, c2_w, c2_b,
           w_ih, w_hh, b_ih, b_hh, w0_w, w0_b, attn_w, attn_b,
           l1_w, l1_b, l2_w, l2_b, h0, c0):
    f32 = jnp.float32
    x_pad = jnp.pad(x, ((0, NPAD - N), (0, 0)))
    batch_pad = jnp.pad(batch, (0, NPAD - N),
                        constant_values=B).reshape(1, NPAD)
    row = edge_index[0]
    col = edge_index[1]
    row3 = (row[None, :] + (jnp.arange(3, dtype=jnp.int32)
                            * NPAD)[:, None]).reshape(-1)
    col3 = col

    degp = _deg_call(col)
    enc_h, z1, dinv_b = _ka(x_pad, degp, lin_w, lin_b.reshape(1, HID), c1_w)
    p1 = _scatter_call(row3, col3, z1.reshape(3 * NPAD, HID))
    p1 = p1.reshape(NC, 3, NPAD, HID)
    out1, z2 = _kb(dinv_b, enc_h, z1, p1, c1_b.reshape(1, HID), c2_w)
    p2 = _scatter_call(row3, col3, z2.reshape(3 * NPAD, HID))
    p2 = p2.reshape(NC, 3, NPAD, HID)
    (seq,) = _kc(dinv_b, out1, z2, p2, c2_b.reshape(1, HID), batch_pad)

    attn_p, out_p = _kd(
        seq, cov.astype(f32), w_ih, w_hh,
        b_ih.reshape(1, 4 * HID), b_hh.reshape(1, 4 * HID),
        w0_w, w0_b.reshape(1, HID),
        attn_w[:, :HID], attn_w[:, HID:], attn_b.reshape(1, 1),
        l1_w, l1_b.reshape(1, 8),
        l2_w[:, :8], l2_w[:, 8:], l2_b.reshape(1, 2),
        h0[0], c0[0])
    return (attn_p[:, :3], out_p[:, :2])


# speedup vs baseline: 2.8319x; 1.2036x over previous
import functools

import jax
import jax.numpy as jnp
from jax import lax
from jax.experimental import pallas as pl
from jax.experimental.pallas import tpu as pltpu
from jax.experimental.pallas import tpu_sc as plsc

N = 10000
NPAD = 10240
E = 320000
B = 64
HID = 128
NCOV = 8
NC, NS, L = 2, 16, 16
NW = NC * NS
EPW = E // NW
CH = 80
EPWP = EPW
CPW = EPWP // CH
CPAIR = 62
STRIPE = NPAD // NS
RB = 1280
GR = NPAD // RB


def _sc_mesh():
    return plsc.VectorSubcoreMesh(core_axis_name="c", subcore_axis_name="s",
                                  num_cores=NC, num_subcores=NS)


@functools.cache
def _deg_kernel():
    return functools.partial(
        pl.kernel,
        out_type=jax.ShapeDtypeStruct((NW, NPAD), jnp.float32),
        mesh=_sc_mesh(),
        compiler_params=pltpu.CompilerParams(needs_layout_passes=False),
        scratch_types=[
            pltpu.VMEM((EPW,), jnp.int32),
            pltpu.VMEM((NPAD,), jnp.float32),
        ],
    )(_deg_body)


def _deg_call(col):
    return _deg_kernel()(col)


def _deg_body(col_hbm, out_hbm, colv, acc):
    cid = lax.axis_index("c")
    sid = lax.axis_index("s")
    wid = cid * NS + sid
    pltpu.sync_copy(col_hbm.at[pl.ds(wid * EPW, EPW)], colv)
    zeros = jnp.zeros((L,), jnp.float32)
    ones = jnp.ones((L,), jnp.float32)

    def zbody(i, _):
        acc[pl.ds(i * L, L)] = zeros
        return _

    lax.fori_loop(0, NPAD // L, zbody, None)

    def hbody(i, _):
        idx = colv[pl.ds(i * L, L)]
        plsc.addupdate_scatter(acc, [idx], ones)
        return _

    lax.fori_loop(0, EPW // L, hbody, None)
    pltpu.sync_copy(acc, out_hbm.at[wid])


@functools.cache
def _scatter_kernel():
    return functools.partial(
        pl.kernel,
        out_type=jax.ShapeDtypeStruct((NC * 3 * NPAD, HID), jnp.float32),
        mesh=_sc_mesh(),
        compiler_params=pltpu.CompilerParams(needs_layout_passes=False),
        scratch_types=[
            pltpu.VMEM((4, CH), jnp.int32),
            pltpu.VMEM((4, CH), jnp.int32),
            pltpu.VMEM((4, CH, HID), jnp.float32),
            pltpu.VMEM_SHARED((NPAD, HID), jnp.float32),
            pltpu.SemaphoreType.DMA((4,)),
            pltpu.SemaphoreType.DMA((4,)),
        ],
    )(_scatter_body)


def _scatter_call(row3, col, z):
    return _scatter_kernel()(row3, col, z)


def _scatter_body(row3_hbm, col_hbm, z_hbm, out_hbm,
                  rowv, colv, gbuf, acc_sh, gsem, ssem):
    cid = lax.axis_index("c")
    sid = lax.axis_index("s")
    wid = cid * NS + sid
    ebase = wid * EPW
    zeros = jnp.zeros((L,), jnp.float32)

    for s in range(3):
        def zb(t, _):
            gbuf[0, t // 8, pl.ds((t % 8) * L, L)] = zeros
            return _

        lax.fori_loop(0, CH * HID // L, zb, None)
        for k in range(STRIPE // CH):
            pltpu.sync_copy(gbuf.at[0],
                            acc_sh.at[pl.ds(sid * STRIPE + k * CH, CH)])
        plsc.subcore_barrier()

        sbase = s * E + ebase

        def load_fire(e, j):
            pltpu.sync_copy(row3_hbm.at[pl.ds(sbase + e * CH, CH)],
                            rowv.at[j])
            pltpu.async_copy(z_hbm.at[rowv.at[j]], gbuf.at[j], gsem.at[j])

        def do_scatter(e, k, sync=False):
            pltpu.make_async_copy(
                z_hbm.at[rowv.at[k]], gbuf.at[k], gsem.at[k]).wait()
            pltpu.sync_copy(col_hbm.at[pl.ds(ebase + e * CH, CH)],
                            colv.at[k])
            if sync:
                pltpu.sync_copy(gbuf.at[k], acc_sh.at[colv.at[k]], add=True)
            else:
                pltpu.async_copy(gbuf.at[k], acc_sh.at[colv.at[k]],
                                 ssem.at[k], add=True)

        def drain(k):
            pltpu.make_async_copy(
                gbuf.at[k], acc_sh.at[colv.at[k]], ssem.at[k]).wait()

        load_fire(0, 0)
        load_fire(1, 1)
        do_scatter(0, 0)
        load_fire(2, 2)
        do_scatter(1, 1)
        load_fire(3, 3)

        def quad(g, _):
            for k in range(4):
                e = 4 * g + 2 + k
                slot = (2 + k) % 4
                do_scatter(e, slot)
                j = (slot + 2) % 4
                drain(j)
                load_fire(e + 2, j)
            return _

        lax.fori_loop(0, 30, quad, None)
        do_scatter(122, 2)
        drain(0)
        load_fire(124, 0)
        do_scatter(123, 3)
        do_scatter(124, 0, sync=True)
        drain(1)
        drain(2)
        drain(3)

        plsc.subcore_barrier()
        obase = (cid * 3 + s) * NPAD + sid * STRIPE
        pltpu.sync_copy(acc_sh.at[pl.ds(sid * STRIPE, STRIPE)],
                        out_hbm.at[pl.ds(obase, STRIPE)])


def _nt(a, b):
    return lax.dot_general(a, b, (((1,), (1,)), ((), ())),
                           preferred_element_type=jnp.float32)


def _ka_body(x_ref, degp_ref, lin_w_ref, lin_b_ref, c1_w_ref,
             h_ref, z1_ref, dinv_ref):
    deg = jnp.sum(degp_ref[...], axis=0, keepdims=True) + 2.0
    dlane = lax.rsqrt(deg)
    ones = jnp.ones((1, HID), jnp.float32)
    dinv = lax.dot_general(dlane, ones, (((0,), (0,)), ((), ())),
                           preferred_element_type=jnp.float32)
    dinv_ref[...] = dinv
    for s in range(3):
        xs = x_ref[:, s * HID:(s + 1) * HID]
        hs = _nt(xs, lin_w_ref[...]) + lin_b_ref[...]
        xw = _nt(hs, c1_w_ref[...])
        h_ref[s] = hs
        z1_ref[s] = dinv * xw


def _kb_body(dinv_ref, h_ref, z1_ref, p_ref, c1_b_ref, c2_w_ref,
             out1_ref, z2_ref):
    dinv = dinv_ref[...]
    agg = p_ref[0, 0] + p_ref[1, 0]
    conv1 = dinv * agg + 2.0 * dinv * z1_ref[0] + c1_b_ref[...]
    o1 = jax.nn.relu(conv1) + h_ref[0]
    out1_ref[0] = o1
    z2_ref[0] = dinv * _nt(o1, c2_w_ref[...])


def _kc_body(dinv_ref, out1_ref, z2_ref, p_ref, c2_b_ref, batch_ref,
             seq_ref, pooled, cnt):
    i = pl.program_id(1)
    dinv = dinv_ref[...]
    agg = p_ref[0, 0] + p_ref[1, 0]
    o2 = dinv * agg + 2.0 * dinv * z2_ref[0] + c2_b_ref[...] + out1_ref[0]
    bt = batch_ref[...]
    ohT = (jnp.broadcast_to(bt, (B, RB))
           == lax.broadcasted_iota(jnp.int32, (B, RB), 0)).astype(jnp.float32)

    @pl.when(i == 0)
    def _():
        pooled[...] = jnp.zeros_like(pooled)
        cnt[...] = jnp.zeros_like(cnt)

    pooled[...] += lax.dot_general(ohT, o2, (((1,), (0,)), ((), ())),
                                   preferred_element_type=jnp.float32)
    cnt[...] += jnp.sum(ohT, axis=1, keepdims=True)

    @pl.when(i == GR - 1)
    def _():
        seq_ref[0] = pooled[...] / jnp.maximum(cnt[...], 1.0)


def _kd_body(seq_ref, cov_ref, w_ih_ref, w_hh_ref, b_ih_ref, b_hh_ref,
             w0_w_ref, w0_b_ref, aw1_ref, aw2_ref, ab_ref,
             l1_w_ref, l1_b_ref, l2a_ref, l2b_ref, l2_b_ref,
             h0_ref, c0_ref, attn_ref, out_ref):
    h = h0_ref[...]
    c = c0_ref[...]
    hs = []
    for t in range(3):
        xt = seq_ref[t]
        g = (_nt(xt, w_ih_ref[...]) + b_ih_ref[...]
             + _nt(h, w_hh_ref[...]) + b_hh_ref[...])
        ii = jax.nn.sigmoid(g[:, 0 * HID:1 * HID])
        ff = jax.nn.sigmoid(g[:, 1 * HID:2 * HID])
        gg = jnp.tanh(g[:, 2 * HID:3 * HID])
        oo = jax.nn.sigmoid(g[:, 3 * HID:4 * HID])
        c = ff * c + ii * gg
        h = oo * jnp.tanh(c)
        hs.append(h)
    hT = hs[-1]
    aw1 = aw1_ref[...]
    aw2 = aw2_ref[...]
    ab = ab_ref[0, 0]
    rs, ws = [], []
    for t in range(3):
        rt = jnp.tanh(_nt(hs[t], w0_w_ref[...]) + w0_b_ref[...])
        wt = (jnp.sum(rt * aw1, axis=1, keepdims=True)
              + jnp.sum(hT * aw2, axis=1, keepdims=True) + ab)
        rs.append(rt)
        ws.append(wt)
    m = jnp.maximum(ws[0], jnp.maximum(ws[1], ws[2]))
    es = [jnp.exp(w - m) for w in ws]
    tot = es[0] + es[1] + es[2]
    als = [e / tot for e in es]
    feat = als[0] * rs[0] + als[1] * rs[1] + als[2] * rs[2]
    l1o = jax.nn.relu(_nt(feat, l1_w_ref[...]) + l1_b_ref[...])
    out = (_nt(l1o, l2a_ref[...]) + _nt(cov_ref[...], l2b_ref[...])
           + l2_b_ref[...])
    attn_ref[...] = jnp.concatenate(
        [als[0], als[1], als[2], jnp.zeros((B, HID - 3), jnp.float32)], axis=1)
    out_ref[...] = jnp.concatenate(
        [out, jnp.zeros((B, HID - 2), jnp.float32)], axis=1)


def _full(shape):
    return pl.BlockSpec(shape, lambda *_: tuple(0 for _ in shape))


_ka = pl.pallas_call(
    _ka_body,
    grid=(GR,),
    in_specs=[
        pl.BlockSpec((RB, 3 * HID), lambda i: (i, 0)),
        pl.BlockSpec((NW, RB), lambda i: (0, i)),
        _full((HID, HID)),
        _full((1, HID)),
        _full((HID, HID)),
    ],
    out_specs=[
        pl.BlockSpec((3, RB, HID), lambda i: (0, i, 0)),
        pl.BlockSpec((3, RB, HID), lambda i: (0, i, 0)),
        pl.BlockSpec((RB, HID), lambda i: (i, 0)),
    ],
    out_shape=[
        jax.ShapeDtypeStruct((3, NPAD, HID), jnp.float32),
        jax.ShapeDtypeStruct((3, NPAD, HID), jnp.float32),
        jax.ShapeDtypeStruct((NPAD, HID), jnp.float32),
    ],
)

_kb = pl.pallas_call(
    _kb_body,
    grid=(3, GR),
    in_specs=[
        pl.BlockSpec((RB, HID), lambda s, i: (i, 0)),
        pl.BlockSpec((1, RB, HID), lambda s, i: (s, i, 0)),
        pl.BlockSpec((1, RB, HID), lambda s, i: (s, i, 0)),
        pl.BlockSpec((NC, 1, RB, HID), lambda s, i: (0, s, i, 0)),
        _full((1, HID)),
        _full((HID, HID)),
    ],
    out_specs=[
        pl.BlockSpec((1, RB, HID), lambda s, i: (s, i, 0)),
        pl.BlockSpec((1, RB, HID), lambda s, i: (s, i, 0)),
    ],
    out_shape=[
        jax.ShapeDtypeStruct((3, NPAD, HID), jnp.float32),
        jax.ShapeDtypeStruct((3, NPAD, HID), jnp.float32),
    ],
)

_kc = pl.pallas_call(
    _kc_body,
    grid=(3, GR),
    in_specs=[
        pl.BlockSpec((RB, HID), lambda s, i: (i, 0)),
        pl.BlockSpec((1, RB, HID), lambda s, i: (s, i, 0)),
        pl.BlockSpec((1, RB, HID), lambda s, i: (s, i, 0)),
        pl.BlockSpec((NC, 1, RB, HID), lambda s, i: (0, s, i, 0)),
        _full((1, HID)),
        pl.BlockSpec((1, RB), lambda s, i: (0, i)),
    ],
    out_specs=[pl.BlockSpec((1, B, HID), lambda s, i: (s, 0, 0))],
    out_shape=[jax.ShapeDtypeStruct((3, B, HID), jnp.float32)],
    scratch_shapes=[
        pltpu.VMEM((B, HID), jnp.float32),
        pltpu.VMEM((B, 1), jnp.float32),
    ],
)

_kd = pl.pallas_call(
    _kd_body,
    out_shape=[
        jax.ShapeDtypeStruct((B, HID), jnp.float32),
        jax.ShapeDtypeStruct((B, HID), jnp.float32),
    ],
)


def kernel(x, edge_index, cov, batch, lin_w, lin_b, c1_w, c1_b, c2_w, c2_b,
           w_ih, w_hh, b_ih, b_hh, w0_w, w0_b, attn_w, attn_b,
           l1_w, l1_b, l2_w, l2_b, h0, c0):
    f32 = jnp.float32
    x_pad = jnp.pad(x, ((0, NPAD - N), (0, 0)))
    batch_pad = jnp.pad(batch, (0, NPAD - N),
                        constant_values=B).reshape(1, NPAD)
    row = edge_index[0]
    col = edge_index[1]
    row3 = (row[None, :] + (jnp.arange(3, dtype=jnp.int32)
                            * NPAD)[:, None]).reshape(-1)
    col3 = col

    degp = _deg_call(col)
    enc_h, z1, dinv_b = _ka(x_pad, degp, lin_w, lin_b.reshape(1, HID), c1_w)
    p1 = _scatter_call(row3, col3, z1.reshape(3 * NPAD, HID))
    p1 = p1.reshape(NC, 3, NPAD, HID)
    out1, z2 = _kb(dinv_b, enc_h, z1, p1, c1_b.reshape(1, HID), c2_w)
    p2 = _scatter_call(row3, col3, z2.reshape(3 * NPAD, HID))
    p2 = p2.reshape(NC, 3, NPAD, HID)
    (seq,) = _kc(dinv_b, out1, z2, p2, c2_b.reshape(1, HID), batch_pad)

    attn_p, out_p = _kd(
        seq, cov.astype(f32), w_ih, w_hh,
        b_ih.reshape(1, 4 * HID), b_hh.reshape(1, 4 * HID),
        w0_w, w0_b.reshape(1, HID),
        attn_w[:, :HID], attn_w[:, HID:], attn_b.reshape(1, 1),
        l1_w, l1_b.reshape(1, 8),
        l2_w[:, :8], l2_w[:, 8:], l2_b.reshape(1, 2),
        h0[0], c0[0])
    return (attn_p[:, :3], out_p[:, :2])


# speedup vs baseline: 3.3302x; 1.1760x over previous
import functools

import jax
import jax.numpy as jnp
from jax import lax
from jax.experimental import pallas as pl
from jax.experimental.pallas import tpu as pltpu
from jax.experimental.pallas import tpu_sc as plsc

N = 10000
NPAD = 10240
E = 320000
B = 64
HID = 128
NCOV = 8
NC, NS, L = 2, 16, 16
NW = NC * NS
EPW = E // NW
CH = 80
EPWP = EPW
CPW = EPWP // CH
CPAIR = 62
STRIPE = NPAD // NS
RB = 1280
GR = NPAD // RB


def _sc_mesh():
    return plsc.VectorSubcoreMesh(core_axis_name="c", subcore_axis_name="s",
                                  num_cores=NC, num_subcores=NS)


@functools.cache
def _deg_kernel():
    return functools.partial(
        pl.kernel,
        out_type=jax.ShapeDtypeStruct((NW, NPAD), jnp.float32),
        mesh=_sc_mesh(),
        compiler_params=pltpu.CompilerParams(needs_layout_passes=False),
        scratch_types=[
            pltpu.VMEM((EPW,), jnp.int32),
            pltpu.VMEM((NPAD,), jnp.float32),
        ],
    )(_deg_body)


def _deg_call(col):
    return _deg_kernel()(col)


def _deg_body(col_hbm, out_hbm, colv, acc):
    cid = lax.axis_index("c")
    sid = lax.axis_index("s")
    wid = cid * NS + sid
    pltpu.sync_copy(col_hbm.at[pl.ds(wid * EPW, EPW)], colv)
    zeros = jnp.zeros((L,), jnp.float32)
    ones = jnp.ones((L,), jnp.float32)

    def zbody(i, _):
        acc[pl.ds(i * L, L)] = zeros
        return _

    lax.fori_loop(0, NPAD // L, zbody, None)

    def hbody(i, _):
        idx = colv[pl.ds(i * L, L)]
        plsc.addupdate_scatter(acc, [idx], ones)
        return _

    lax.fori_loop(0, EPW // L, hbody, None)
    pltpu.sync_copy(acc, out_hbm.at[wid])


@functools.cache
def _scatter_kernel():
    return functools.partial(
        pl.kernel,
        out_type=jax.ShapeDtypeStruct((NC * 3 * NPAD, HID), jnp.float32),
        mesh=_sc_mesh(),
        compiler_params=pltpu.CompilerParams(needs_layout_passes=False),
        scratch_types=[
            pltpu.VMEM((4, CH), jnp.int32),
            pltpu.VMEM((4, CH), jnp.int32),
            pltpu.VMEM((4, CH, HID), jnp.float32),
            pltpu.VMEM_SHARED((NPAD, HID), jnp.float32),
            pltpu.SemaphoreType.DMA((4,)),
            pltpu.SemaphoreType.DMA((4,)),
            pltpu.SemaphoreType.DMA((4,)),
        ],
    )(_scatter_body)


def _scatter_call(row3, col, z):
    return _scatter_kernel()(row3, col, z)


def _scatter_body(row3_hbm, col_hbm, z_hbm, out_hbm,
                  rowv, colv, gbuf, acc_sh, gsem, ssem, csem):
    cid = lax.axis_index("c")
    sid = lax.axis_index("s")
    wid = cid * NS + sid
    ebase = wid * EPW
    zeros = jnp.zeros((L,), jnp.float32)

    for s in range(3):
        def zb(t, _):
            gbuf[0, t // 8, pl.ds((t % 8) * L, L)] = zeros
            return _

        lax.fori_loop(0, CH * HID // L, zb, None)
        for k in range(STRIPE // CH):
            pltpu.sync_copy(gbuf.at[0],
                            acc_sh.at[pl.ds(sid * STRIPE + k * CH, CH)])
        plsc.subcore_barrier()

        sbase = s * E + ebase

        def load_fire(e, j):
            pltpu.sync_copy(row3_hbm.at[pl.ds(sbase + e * CH, CH)],
                            rowv.at[j])
            pltpu.async_copy(z_hbm.at[rowv.at[j]], gbuf.at[j], gsem.at[j])

        def col_prefetch(e, k):
            pltpu.async_copy(col_hbm.at[pl.ds(ebase + e * CH, CH)],
                             colv.at[k], csem.at[k])

        def do_scatter(e, k, sync=False):
            pltpu.make_async_copy(
                z_hbm.at[rowv.at[k]], gbuf.at[k], gsem.at[k]).wait()
            pltpu.make_async_copy(
                col_hbm.at[pl.ds(ebase, CH)], colv.at[k], csem.at[k]).wait()
            if sync:
                pltpu.sync_copy(gbuf.at[k], acc_sh.at[colv.at[k]], add=True)
            else:
                pltpu.async_copy(gbuf.at[k], acc_sh.at[colv.at[k]],
                                 ssem.at[k], add=True)

        def drain(k):
            pltpu.make_async_copy(
                gbuf.at[k], acc_sh.at[colv.at[k]], ssem.at[k]).wait()

        for j in range(4):
            col_prefetch(j, j)
        load_fire(0, 0)
        load_fire(1, 1)
        do_scatter(0, 0)
        load_fire(2, 2)
        do_scatter(1, 1)
        load_fire(3, 3)

        def quad(g, _):
            for k in range(4):
                e = 4 * g + 2 + k
                slot = (2 + k) % 4
                do_scatter(e, slot)
                j = (slot + 2) % 4
                drain(j)
                col_prefetch(e + 2, j)
                load_fire(e + 2, j)
            return _

        lax.fori_loop(0, 30, quad, None)
        do_scatter(122, 2)
        drain(0)
        col_prefetch(124, 0)
        load_fire(124, 0)
        do_scatter(123, 3)
        do_scatter(124, 0, sync=True)
        drain(1)
        drain(2)
        drain(3)

        plsc.subcore_barrier()
        obase = (cid * 3 + s) * NPAD + sid * STRIPE
        pltpu.sync_copy(acc_sh.at[pl.ds(sid * STRIPE, STRIPE)],
                        out_hbm.at[pl.ds(obase, STRIPE)])


def _nt(a, b):
    return lax.dot_general(a, b, (((1,), (1,)), ((), ())),
                           preferred_element_type=jnp.float32)


def _ka_body(x_ref, degp_ref, lin_w_ref, lin_b_ref, c1_w_ref,
             h_ref, z1_ref, dinv_ref):
    deg = jnp.sum(degp_ref[...], axis=0, keepdims=True) + 2.0
    dlane = lax.rsqrt(deg)
    ones = jnp.ones((1, HID), jnp.float32)
    dinv = lax.dot_general(dlane, ones, (((0,), (0,)), ((), ())),
                           preferred_element_type=jnp.float32)
    dinv_ref[...] = dinv
    for s in range(3):
        xs = x_ref[:, s * HID:(s + 1) * HID]
        hs = _nt(xs, lin_w_ref[...]) + lin_b_ref[...]
        xw = _nt(hs, c1_w_ref[...])
        h_ref[s] = hs
        z1_ref[s] = dinv * xw


def _kb_body(dinv_ref, h_ref, z1_ref, p_ref, c1_b_ref, c2_w_ref,
             out1_ref, z2_ref):
    dinv = dinv_ref[...]
    agg = p_ref[0, 0] + p_ref[1, 0]
    conv1 = dinv * agg + 2.0 * dinv * z1_ref[0] + c1_b_ref[...]
    o1 = jax.nn.relu(conv1) + h_ref[0]
    out1_ref[0] = o1
    z2_ref[0] = dinv * _nt(o1, c2_w_ref[...])


def _kc_body(dinv_ref, out1_ref, z2_ref, p_ref, c2_b_ref, batch_ref,
             seq_ref, pooled, cnt):
    i = pl.program_id(1)
    dinv = dinv_ref[...]
    agg = p_ref[0, 0] + p_ref[1, 0]
    o2 = dinv * agg + 2.0 * dinv * z2_ref[0] + c2_b_ref[...] + out1_ref[0]
    bt = batch_ref[...]
    ohT = (jnp.broadcast_to(bt, (B, RB))
           == lax.broadcasted_iota(jnp.int32, (B, RB), 0)).astype(jnp.float32)

    @pl.when(i == 0)
    def _():
        pooled[...] = jnp.zeros_like(pooled)
        cnt[...] = jnp.zeros_like(cnt)

    pooled[...] += lax.dot_general(ohT, o2, (((1,), (0,)), ((), ())),
                                   preferred_element_type=jnp.float32)
    cnt[...] += jnp.sum(ohT, axis=1, keepdims=True)

    @pl.when(i == GR - 1)
    def _():
        seq_ref[0] = pooled[...] / jnp.maximum(cnt[...], 1.0)


def _kd_body(seq_ref, cov_ref, w_ih_ref, w_hh_ref, b_ih_ref, b_hh_ref,
             w0_w_ref, w0_b_ref, aw1_ref, aw2_ref, ab_ref,
             l1_w_ref, l1_b_ref, l2a_ref, l2b_ref, l2_b_ref,
             h0_ref, c0_ref, attn_ref, out_ref):
    h = h0_ref[...]
    c = c0_ref[...]
    hs = []
    for t in range(3):
        xt = seq_ref[t]
        g = (_nt(xt, w_ih_ref[...]) + b_ih_ref[...]
             + _nt(h, w_hh_ref[...]) + b_hh_ref[...])
        ii = jax.nn.sigmoid(g[:, 0 * HID:1 * HID])
        ff = jax.nn.sigmoid(g[:, 1 * HID:2 * HID])
        gg = jnp.tanh(g[:, 2 * HID:3 * HID])
        oo = jax.nn.sigmoid(g[:, 3 * HID:4 * HID])
        c = ff * c + ii * gg
        h = oo * jnp.tanh(c)
        hs.append(h)
    hT = hs[-1]
    aw1 = aw1_ref[...]
    aw2 = aw2_ref[...]
    ab = ab_ref[0, 0]
    rs, ws = [], []
    for t in range(3):
        rt = jnp.tanh(_nt(hs[t], w0_w_ref[...]) + w0_b_ref[...])
        wt = (jnp.sum(rt * aw1, axis=1, keepdims=True)
              + jnp.sum(hT * aw2, axis=1, keepdims=True) + ab)
        rs.append(rt)
        ws.append(wt)
    m = jnp.maximum(ws[0], jnp.maximum(ws[1], ws[2]))
    es = [jnp.exp(w - m) for w in ws]
    tot = es[0] + es[1] + es[2]
    als = [e / tot for e in es]
    feat = als[0] * rs[0] + als[1] * rs[1] + als[2] * rs[2]
    l1o = jax.nn.relu(_nt(feat, l1_w_ref[...]) + l1_b_ref[...])
    out = (_nt(l1o, l2a_ref[...]) + _nt(cov_ref[...], l2b_ref[...])
           + l2_b_ref[...])
    attn_ref[...] = jnp.concatenate(
        [als[0], als[1], als[2], jnp.zeros((B, HID - 3), jnp.float32)], axis=1)
    out_ref[...] = jnp.concatenate(
        [out, jnp.zeros((B, HID - 2), jnp.float32)], axis=1)


def _full(shape):
    return pl.BlockSpec(shape, lambda *_: tuple(0 for _ in shape))


_ka = pl.pallas_call(
    _ka_body,
    grid=(GR,),
    in_specs=[
        pl.BlockSpec((RB, 3 * HID), lambda i: (i, 0)),
        pl.BlockSpec((NW, RB), lambda i: (0, i)),
        _full((HID, HID)),
        _full((1, HID)),
        _full((HID, HID)),
    ],
    out_specs=[
        pl.BlockSpec((3, RB, HID), lambda i: (0, i, 0)),
        pl.BlockSpec((3, RB, HID), lambda i: (0, i, 0)),
        pl.BlockSpec((RB, HID), lambda i: (i, 0)),
    ],
    out_shape=[
        jax.ShapeDtypeStruct((3, NPAD, HID), jnp.float32),
        jax.ShapeDtypeStruct((3, NPAD, HID), jnp.float32),
        jax.ShapeDtypeStruct((NPAD, HID), jnp.float32),
    ],
)

_kb = pl.pallas_call(
    _kb_body,
    grid=(3, GR),
    in_specs=[
        pl.BlockSpec((RB, HID), lambda s, i: (i, 0)),
        pl.BlockSpec((1, RB, HID), lambda s, i: (s, i, 0)),
        pl.BlockSpec((1, RB, HID), lambda s, i: (s, i, 0)),
        pl.BlockSpec((NC, 1, RB, HID), lambda s, i: (0, s, i, 0)),
        _full((1, HID)),
        _full((HID, HID)),
    ],
    out_specs=[
        pl.BlockSpec((1, RB, HID), lambda s, i: (s, i, 0)),
        pl.BlockSpec((1, RB, HID), lambda s, i: (s, i, 0)),
    ],
    out_shape=[
        jax.ShapeDtypeStruct((3, NPAD, HID), jnp.float32),
        jax.ShapeDtypeStruct((3, NPAD, HID), jnp.float32),
    ],
)

_kc = pl.pallas_call(
    _kc_body,
    grid=(3, GR),
    in_specs=[
        pl.BlockSpec((RB, HID), lambda s, i: (i, 0)),
        pl.BlockSpec((1, RB, HID), lambda s, i: (s, i, 0)),
        pl.BlockSpec((1, RB, HID), lambda s, i: (s, i, 0)),
        pl.BlockSpec((NC, 1, RB, HID), lambda s, i: (0, s, i, 0)),
        _full((1, HID)),
        pl.BlockSpec((1, RB), lambda s, i: (0, i)),
    ],
    out_specs=[pl.BlockSpec((1, B, HID), lambda s, i: (s, 0, 0))],
    out_shape=[jax.ShapeDtypeStruct((3, B, HID), jnp.float32)],
    scratch_shapes=[
        pltpu.VMEM((B, HID), jnp.float32),
        pltpu.VMEM((B, 1), jnp.float32),
    ],
)

_kd = pl.pallas_call(
    _kd_body,
    out_shape=[
        jax.ShapeDtypeStruct((B, HID), jnp.float32),
        jax.ShapeDtypeStruct((B, HID), jnp.float32),
    ],
)


def kernel(x, edge_index, cov, batch, lin_w, lin_b, c1_w, c1_b, c2_w, c2_b,
           w_ih, w_hh, b_ih, b_hh, w0_w, w0_b, attn_w, attn_b,
           l1_w, l1_b, l2_w, l2_b, h0, c0):
    f32 = jnp.float32
    x_pad = jnp.pad(x, ((0, NPAD - N), (0, 0)))
    batch_pad = jnp.pad(batch, (0, NPAD - N),
                        constant_values=B).reshape(1, NPAD)
    row = edge_index[0]
    col = edge_index[1]
    row3 = (row[None, :] + (jnp.arange(3, dtype=jnp.int32)
                            * NPAD)[:, None]).reshape(-1)
    col3 = col

    degp = _deg_call(col)
    enc_h, z1, dinv_b = _ka(x_pad, degp, lin_w, lin_b.reshape(1, HID), c1_w)
    p1 = _scatter_call(row3, col3, z1.reshape(3 * NPAD, HID))
    p1 = p1.reshape(NC, 3, NPAD, HID)
    out1, z2 = _kb(dinv_b, enc_h, z1, p1, c1_b.reshape(1, HID), c2_w)
    p2 = _scatter_call(row3, col3, z2.reshape(3 * NPAD, HID))
    p2 = p2.reshape(NC, 3, NPAD, HID)
    (seq,) = _kc(dinv_b, out1, z2, p2, c2_b.reshape(1, HID), batch_pad)

    attn_p, out_p = _kd(
        seq, cov.astype(f32), w_ih, w_hh,
        b_ih.reshape(1, 4 * HID), b_hh.reshape(1, 4 * HID),
        w0_w, w0_b.reshape(1, HID),
        attn_w[:, :HID], attn_w[:, HID:], attn_b.reshape(1, 1),
        l1_w, l1_b.reshape(1, 8),
        l2_w[:, :8], l2_w[:, 8:], l2_b.reshape(1, 2),
        h0[0], c0[0])
    return (attn_p[:, :3], out_p[:, :2])


# speedup vs baseline: 3.4997x; 1.0509x over previous
import functools

import jax
import jax.numpy as jnp
from jax import lax
from jax.experimental import pallas as pl
from jax.experimental.pallas import tpu as pltpu
from jax.experimental.pallas import tpu_sc as plsc

N = 10000
NPAD = 10240
E = 320000
B = 64
HID = 128
NCOV = 8
NC, NS, L = 2, 16, 16
NW = NC * NS
EPW = E // NW
CH = 80
EPWP = EPW
CPW = EPWP // CH
CPAIR = 62
STRIPE = NPAD // NS
RB = 1280
GR = NPAD // RB


def _sc_mesh():
    return plsc.VectorSubcoreMesh(core_axis_name="c", subcore_axis_name="s",
                                  num_cores=NC, num_subcores=NS)


@functools.cache
def _deg_kernel():
    return functools.partial(
        pl.kernel,
        out_type=jax.ShapeDtypeStruct((NW, NPAD), jnp.float32),
        mesh=_sc_mesh(),
        compiler_params=pltpu.CompilerParams(needs_layout_passes=False),
        scratch_types=[
            pltpu.VMEM((EPW,), jnp.int32),
            pltpu.VMEM((NPAD,), jnp.float32),
        ],
    )(_deg_body)


def _deg_call(col):
    return _deg_kernel()(col)


def _deg_body(col_hbm, out_hbm, colv, acc):
    cid = lax.axis_index("c")
    sid = lax.axis_index("s")
    wid = cid * NS + sid
    pltpu.sync_copy(col_hbm.at[pl.ds(wid * EPW, EPW)], colv)
    zeros = jnp.zeros((L,), jnp.float32)
    ones = jnp.ones((L,), jnp.float32)

    def zbody(i, _):
        acc[pl.ds(i * L, L)] = zeros
        return _

    lax.fori_loop(0, NPAD // L, zbody, None)

    def hbody(i, _):
        idx = colv[pl.ds(i * L, L)]
        plsc.addupdate_scatter(acc, [idx], ones)
        return _

    lax.fori_loop(0, EPW // L, hbody, None)
    pltpu.sync_copy(acc, out_hbm.at[wid])


@functools.cache
def _scatter_kernel():
    return functools.partial(
        pl.kernel,
        out_type=jax.ShapeDtypeStruct((NC * 3 * NPAD, HID), jnp.float32),
        mesh=_sc_mesh(),
        compiler_params=pltpu.CompilerParams(needs_layout_passes=False),
        scratch_types=[
            pltpu.VMEM((4, CH), jnp.int32),
            pltpu.VMEM((4, CH), jnp.int32),
            pltpu.VMEM((4, CH, HID), jnp.float32),
            pltpu.VMEM_SHARED((NPAD, HID), jnp.float32),
            pltpu.SemaphoreType.DMA((4,)),
            pltpu.SemaphoreType.DMA((4,)),
            pltpu.SemaphoreType.DMA((4,)),
            pltpu.SemaphoreType.DMA((4,)),
        ],
    )(_scatter_body)


def _scatter_call(row3, col, z):
    return _scatter_kernel()(row3, col, z)


def _scatter_body(row3_hbm, col_hbm, z_hbm, out_hbm,
                  rowv, colv, gbuf, acc_sh, gsem, ssem, csem, rsem):
    cid = lax.axis_index("c")
    sid = lax.axis_index("s")
    wid = cid * NS + sid
    ebase = wid * EPW
    zeros = jnp.zeros((L,), jnp.float32)

    for s in range(3):
        def zb(t, _):
            gbuf[0, t // 8, pl.ds((t % 8) * L, L)] = zeros
            return _

        lax.fori_loop(0, CH * HID // L, zb, None)
        for k in range(STRIPE // CH):
            pltpu.sync_copy(gbuf.at[0],
                            acc_sh.at[pl.ds(sid * STRIPE + k * CH, CH)])
        plsc.subcore_barrier()

        sbase = s * E + ebase

        def row_prefetch(e, j):
            pltpu.async_copy(row3_hbm.at[pl.ds(sbase + e * CH, CH)],
                             rowv.at[j], rsem.at[j])

        def col_prefetch(e, k):
            pltpu.async_copy(col_hbm.at[pl.ds(ebase + e * CH, CH)],
                             colv.at[k], csem.at[k])

        def gather_fire(j):
            pltpu.make_async_copy(
                row3_hbm.at[pl.ds(sbase, CH)], rowv.at[j], rsem.at[j]).wait()
            pltpu.async_copy(z_hbm.at[rowv.at[j]], gbuf.at[j], gsem.at[j])

        def do_scatter(e, k, sync=False):
            pltpu.make_async_copy(
                z_hbm.at[rowv.at[k]], gbuf.at[k], gsem.at[k]).wait()
            pltpu.make_async_copy(
                col_hbm.at[pl.ds(ebase, CH)], colv.at[k], csem.at[k]).wait()
            if sync:
                pltpu.sync_copy(gbuf.at[k], acc_sh.at[colv.at[k]], add=True)
            else:
                pltpu.async_copy(gbuf.at[k], acc_sh.at[colv.at[k]],
                                 ssem.at[k], add=True)

        def drain(k):
            pltpu.make_async_copy(
                gbuf.at[k], acc_sh.at[colv.at[k]], ssem.at[k]).wait()

        for j in range(4):
            col_prefetch(j, j)
            row_prefetch(j, j)
        gather_fire(0)
        gather_fire(1)
        do_scatter(0, 0)
        row_prefetch(4, 0)
        gather_fire(2)
        do_scatter(1, 1)
        row_prefetch(5, 1)
        gather_fire(3)

        def quad(g, _):
            for k in range(4):
                e = 4 * g + 2 + k
                slot = (2 + k) % 4
                do_scatter(e, slot)
                row_prefetch(e + 4, slot)
                j = (slot + 2) % 4
                drain(j)
                col_prefetch(e + 2, j)
                gather_fire(j)
            return _

        lax.fori_loop(0, 29, quad, None)
        for e in range(118, 125):
            slot = e % 4
            do_scatter(e, slot, sync=(e == 124))
            if e + 4 <= 124:
                row_prefetch(e + 4, slot)
            if e + 2 <= 124:
                j = (slot + 2) % 4
                drain(j)
                col_prefetch(e + 2, j)
                gather_fire(j)
        drain(1)
        drain(2)
        drain(3)

        plsc.subcore_barrier()
        obase = (cid * 3 + s) * NPAD + sid * STRIPE
        pltpu.sync_copy(acc_sh.at[pl.ds(sid * STRIPE, STRIPE)],
                        out_hbm.at[pl.ds(obase, STRIPE)])


def _nt(a, b):
    return lax.dot_general(a, b, (((1,), (1,)), ((), ())),
                           preferred_element_type=jnp.float32)


def _ka_body(x_ref, degp_ref, lin_w_ref, lin_b_ref, c1_w_ref,
             h_ref, z1_ref, dinv_ref):
    deg = jnp.sum(degp_ref[...], axis=0, keepdims=True) + 2.0
    dlane = lax.rsqrt(deg)
    ones = jnp.ones((1, HID), jnp.float32)
    dinv = lax.dot_general(dlane, ones, (((0,), (0,)), ((), ())),
                           preferred_element_type=jnp.float32)
    dinv_ref[...] = dinv
    for s in range(3):
        xs = x_ref[:, s * HID:(s + 1) * HID]
        hs = _nt(xs, lin_w_ref[...]) + lin_b_ref[...]
        xw = _nt(hs, c1_w_ref[...])
        h_ref[s] = hs
        z1_ref[s] = dinv * xw


def _kb_body(dinv_ref, h_ref, z1_ref, p_ref, c1_b_ref, c2_w_ref,
             out1_ref, z2_ref):
    dinv = dinv_ref[...]
    agg = p_ref[0, 0] + p_ref[1, 0]
    conv1 = dinv * agg + 2.0 * dinv * z1_ref[0] + c1_b_ref[...]
    o1 = jax.nn.relu(conv1) + h_ref[0]
    out1_ref[0] = o1
    z2_ref[0] = dinv * _nt(o1, c2_w_ref[...])


def _kc_body(dinv_ref, out1_ref, z2_ref, p_ref, c2_b_ref, batch_ref,
             seq_ref, pooled, cnt):
    i = pl.program_id(1)
    dinv = dinv_ref[...]
    agg = p_ref[0, 0] + p_ref[1, 0]
    o2 = dinv * agg + 2.0 * dinv * z2_ref[0] + c2_b_ref[...] + out1_ref[0]
    bt = batch_ref[...]
    ohT = (jnp.broadcast_to(bt, (B, RB))
           == lax.broadcasted_iota(jnp.int32, (B, RB), 0)).astype(jnp.float32)

    @pl.when(i == 0)
    def _():
        pooled[...] = jnp.zeros_like(pooled)
        cnt[...] = jnp.zeros_like(cnt)

    pooled[...] += lax.dot_general(ohT, o2, (((1,), (0,)), ((), ())),
                                   preferred_element_type=jnp.float32)
    cnt[...] += jnp.sum(ohT, axis=1, keepdims=True)

    @pl.when(i == GR - 1)
    def _():
        seq_ref[0] = pooled[...] / jnp.maximum(cnt[...], 1.0)


def _kd_body(seq_ref, cov_ref, w_ih_ref, w_hh_ref, b_ih_ref, b_hh_ref,
             w0_w_ref, w0_b_ref, aw1_ref, aw2_ref, ab_ref,
             l1_w_ref, l1_b_ref, l2a_ref, l2b_ref, l2_b_ref,
             h0_ref, c0_ref, attn_ref, out_ref):
    h = h0_ref[...]
    c = c0_ref[...]
    hs = []
    for t in range(3):
        xt = seq_ref[t]
        g = (_nt(xt, w_ih_ref[...]) + b_ih_ref[...]
             + _nt(h, w_hh_ref[...]) + b_hh_ref[...])
        ii = jax.nn.sigmoid(g[:, 0 * HID:1 * HID])
        ff = jax.nn.sigmoid(g[:, 1 * HID:2 * HID])
        gg = jnp.tanh(g[:, 2 * HID:3 * HID])
        oo = jax.nn.sigmoid(g[:, 3 * HID:4 * HID])
        c = ff * c + ii * gg
        h = oo * jnp.tanh(c)
        hs.append(h)
    hT = hs[-1]
    aw1 = aw1_ref[...]
    aw2 = aw2_ref[...]
    ab = ab_ref[0, 0]
    rs, ws = [], []
    for t in range(3):
        rt = jnp.tanh(_nt(hs[t], w0_w_ref[...]) + w0_b_ref[...])
        wt = (jnp.sum(rt * aw1, axis=1, keepdims=True)
              + jnp.sum(hT * aw2, axis=1, keepdims=True) + ab)
        rs.append(rt)
        ws.append(wt)
    m = jnp.maximum(ws[0], jnp.maximum(ws[1], ws[2]))
    es = [jnp.exp(w - m) for w in ws]
    tot = es[0] + es[1] + es[2]
    als = [e / tot for e in es]
    feat = als[0] * rs[0] + als[1] * rs[1] + als[2] * rs[2]
    l1o = jax.nn.relu(_nt(feat, l1_w_ref[...]) + l1_b_ref[...])
    out = (_nt(l1o, l2a_ref[...]) + _nt(cov_ref[...], l2b_ref[...])
           + l2_b_ref[...])
    attn_ref[...] = jnp.concatenate(
        [als[0], als[1], als[2], jnp.zeros((B, HID - 3), jnp.float32)], axis=1)
    out_ref[...] = jnp.concatenate(
        [out, jnp.zeros((B, HID - 2), jnp.float32)], axis=1)


def _full(shape):
    return pl.BlockSpec(shape, lambda *_: tuple(0 for _ in shape))


_ka = pl.pallas_call(
    _ka_body,
    grid=(GR,),
    in_specs=[
        pl.BlockSpec((RB, 3 * HID), lambda i: (i, 0)),
        pl.BlockSpec((NW, RB), lambda i: (0, i)),
        _full((HID, HID)),
        _full((1, HID)),
        _full((HID, HID)),
    ],
    out_specs=[
        pl.BlockSpec((3, RB, HID), lambda i: (0, i, 0)),
        pl.BlockSpec((3, RB, HID), lambda i: (0, i, 0)),
        pl.BlockSpec((RB, HID), lambda i: (i, 0)),
    ],
    out_shape=[
        jax.ShapeDtypeStruct((3, NPAD, HID), jnp.float32),
        jax.ShapeDtypeStruct((3, NPAD, HID), jnp.float32),
        jax.ShapeDtypeStruct((NPAD, HID), jnp.float32),
    ],
)

_kb = pl.pallas_call(
    _kb_body,
    grid=(3, GR),
    in_specs=[
        pl.BlockSpec((RB, HID), lambda s, i: (i, 0)),
        pl.BlockSpec((1, RB, HID), lambda s, i: (s, i, 0)),
        pl.BlockSpec((1, RB, HID), lambda s, i: (s, i, 0)),
        pl.BlockSpec((NC, 1, RB, HID), lambda s, i: (0, s, i, 0)),
        _full((1, HID)),
        _full((HID, HID)),
    ],
    out_specs=[
        pl.BlockSpec((1, RB, HID), lambda s, i: (s, i, 0)),
        pl.BlockSpec((1, RB, HID), lambda s, i: (s, i, 0)),
    ],
    out_shape=[
        jax.ShapeDtypeStruct((3, NPAD, HID), jnp.float32),
        jax.ShapeDtypeStruct((3, NPAD, HID), jnp.float32),
    ],
)

_kc = pl.pallas_call(
    _kc_body,
    grid=(3, GR),
    in_specs=[
        pl.BlockSpec((RB, HID), lambda s, i: (i, 0)),
        pl.BlockSpec((1, RB, HID), lambda s, i: (s, i, 0)),
        pl.BlockSpec((1, RB, HID), lambda s, i: (s, i, 0)),
        pl.BlockSpec((NC, 1, RB, HID), lambda s, i: (0, s, i, 0)),
        _full((1, HID)),
        pl.BlockSpec((1, RB), lambda s, i: (0, i)),
    ],
    out_specs=[pl.BlockSpec((1, B, HID), lambda s, i: (s, 0, 0))],
    out_shape=[jax.ShapeDtypeStruct((3, B, HID), jnp.float32)],
    scratch_shapes=[
        pltpu.VMEM((B, HID), jnp.float32),
        pltpu.VMEM((B, 1), jnp.float32),
    ],
)

_kd = pl.pallas_call(
    _kd_body,
    out_shape=[
        jax.ShapeDtypeStruct((B, HID), jnp.float32),
        jax.ShapeDtypeStruct((B, HID), jnp.float32),
    ],
)


def kernel(x, edge_index, cov, batch, lin_w, lin_b, c1_w, c1_b, c2_w, c2_b,
           w_ih, w_hh, b_ih, b_hh, w0_w, w0_b, attn_w, attn_b,
           l1_w, l1_b, l2_w, l2_b, h0, c0):
    f32 = jnp.float32
    x_pad = jnp.pad(x, ((0, NPAD - N), (0, 0)))
    batch_pad = jnp.pad(batch, (0, NPAD - N),
                        constant_values=B).reshape(1, NPAD)
    row = edge_index[0]
    col = edge_index[1]
    row3 = (row[None, :] + (jnp.arange(3, dtype=jnp.int32)
                            * NPAD)[:, None]).reshape(-1)
    col3 = col

    degp = _deg_call(col)
    enc_h, z1, dinv_b = _ka(x_pad, degp, lin_w, lin_b.reshape(1, HID), c1_w)
    p1 = _scatter_call(row3, col3, z1.reshape(3 * NPAD, HID))
    p1 = p1.reshape(NC, 3, NPAD, HID)
    out1, z2 = _kb(dinv_b, enc_h, z1, p1, c1_b.reshape(1, HID), c2_w)
    p2 = _scatter_call(row3, col3, z2.reshape(3 * NPAD, HID))
    p2 = p2.reshape(NC, 3, NPAD, HID)
    (seq,) = _kc(dinv_b, out1, z2, p2, c2_b.reshape(1, HID), batch_pad)

    attn_p, out_p = _kd(
        seq, cov.astype(f32), w_ih, w_hh,
        b_ih.reshape(1, 4 * HID), b_hh.reshape(1, 4 * HID),
        w0_w, w0_b.reshape(1, HID),
        attn_w[:, :HID], attn_w[:, HID:], attn_b.reshape(1, 1),
        l1_w, l1_b.reshape(1, 8),
        l2_w[:, :8], l2_w[:, 8:], l2_b.reshape(1, 2),
        h0[0], c0[0])
    return (attn_p[:, :3], out_p[:, :2])


# speedup vs baseline: 3.6834x; 1.0525x over previous
import functools

import jax
import jax.numpy as jnp
from jax import lax
from jax.experimental import pallas as pl
from jax.experimental.pallas import tpu as pltpu
from jax.experimental.pallas import tpu_sc as plsc

N = 10000
NPAD = 10240
E = 320000
B = 64
HID = 128
NCOV = 8
NC, NS, L = 2, 16, 16
NW = NC * NS
EPW = E // NW
CH = 80
EPWP = EPW
CPW = EPWP // CH
CPAIR = 62
STRIPE = NPAD // NS
RB = 1280
GR = NPAD // RB


def _sc_mesh():
    return plsc.VectorSubcoreMesh(core_axis_name="c", subcore_axis_name="s",
                                  num_cores=NC, num_subcores=NS)


@functools.cache
def _deg_kernel():
    return functools.partial(
        pl.kernel,
        out_type=jax.ShapeDtypeStruct((NW, NPAD), jnp.float32),
        mesh=_sc_mesh(),
        compiler_params=pltpu.CompilerParams(needs_layout_passes=False),
        scratch_types=[
            pltpu.VMEM((EPW,), jnp.int32),
            pltpu.VMEM((NPAD,), jnp.float32),
        ],
    )(_deg_body)


def _deg_call(col):
    return _deg_kernel()(col)


def _deg_body(col_hbm, out_hbm, colv, acc):
    cid = lax.axis_index("c")
    sid = lax.axis_index("s")
    wid = cid * NS + sid
    pltpu.sync_copy(col_hbm.at[pl.ds(wid * EPW, EPW)], colv)
    zeros = jnp.zeros((L,), jnp.float32)
    ones = jnp.ones((L,), jnp.float32)

    def zbody(i, _):
        acc[pl.ds(i * L, L)] = zeros
        return _

    lax.fori_loop(0, NPAD // L, zbody, None)

    def hbody(i, _):
        idx = colv[pl.ds(i * L, L)]
        plsc.addupdate_scatter(acc, [idx], ones)
        return _

    lax.fori_loop(0, EPW // L, hbody, None)
    pltpu.sync_copy(acc, out_hbm.at[wid])


@functools.cache
def _scatter_kernel():
    return functools.partial(
        pl.kernel,
        out_type=jax.ShapeDtypeStruct((NC * NPAD, HID), jnp.float32),
        mesh=_sc_mesh(),
        compiler_params=pltpu.CompilerParams(needs_layout_passes=False),
        scratch_types=[
            pltpu.VMEM((4, CH), jnp.int32),
            pltpu.VMEM((4, CH), jnp.int32),
            pltpu.VMEM((4, CH, HID), jnp.float32),
            pltpu.VMEM_SHARED((NPAD, HID), jnp.float32),
            pltpu.SemaphoreType.DMA((4,)),
            pltpu.SemaphoreType.DMA((4,)),
            pltpu.SemaphoreType.DMA((4,)),
            pltpu.SemaphoreType.DMA((4,)),
        ],
    )(_scatter_body)


def _scatter_call(row, col, z):
    return _scatter_kernel()(row, col, z)


def _scatter_body(row_hbm, col_hbm, z_hbm, out_hbm,
                  rowv, colv, gbuf, acc_sh, gsem, ssem, csem, rsem):
    cid = lax.axis_index("c")
    sid = lax.axis_index("s")
    wid = cid * NS + sid
    ebase = wid * EPW
    zeros = jnp.zeros((L,), jnp.float32)

    if True:
        def zb(t, _):
            gbuf[0, t // 8, pl.ds((t % 8) * L, L)] = zeros
            return _

        lax.fori_loop(0, CH * HID // L, zb, None)
        for k in range(STRIPE // CH):
            pltpu.sync_copy(gbuf.at[0],
                            acc_sh.at[pl.ds(sid * STRIPE + k * CH, CH)])
        plsc.subcore_barrier()

        sbase = ebase

        def row_prefetch(e, j):
            pltpu.async_copy(row_hbm.at[pl.ds(sbase + e * CH, CH)],
                             rowv.at[j], rsem.at[j])

        def col_prefetch(e, k):
            pltpu.async_copy(col_hbm.at[pl.ds(ebase + e * CH, CH)],
                             colv.at[k], csem.at[k])

        def gather_fire(j):
            pltpu.make_async_copy(
                row_hbm.at[pl.ds(sbase, CH)], rowv.at[j], rsem.at[j]).wait()
            pltpu.async_copy(z_hbm.at[rowv.at[j]], gbuf.at[j], gsem.at[j])

        def do_scatter(e, k, sync=False):
            pltpu.make_async_copy(
                z_hbm.at[rowv.at[k]], gbuf.at[k], gsem.at[k]).wait()
            pltpu.make_async_copy(
                col_hbm.at[pl.ds(ebase, CH)], colv.at[k], csem.at[k]).wait()
            if sync:
                pltpu.sync_copy(gbuf.at[k], acc_sh.at[colv.at[k]], add=True)
            else:
                pltpu.async_copy(gbuf.at[k], acc_sh.at[colv.at[k]],
                                 ssem.at[k], add=True)

        def drain(k):
            pltpu.make_async_copy(
                gbuf.at[k], acc_sh.at[colv.at[k]], ssem.at[k]).wait()

        for j in range(4):
            col_prefetch(j, j)
            row_prefetch(j, j)
        gather_fire(0)
        gather_fire(1)
        do_scatter(0, 0)
        row_prefetch(4, 0)
        gather_fire(2)
        do_scatter(1, 1)
        row_prefetch(5, 1)
        gather_fire(3)

        def quad(g, _):
            for k in range(4):
                e = 4 * g + 2 + k
                slot = (2 + k) % 4
                do_scatter(e, slot)
                row_prefetch(e + 4, slot)
                j = (slot + 2) % 4
                drain(j)
                col_prefetch(e + 2, j)
                gather_fire(j)
            return _

        lax.fori_loop(0, 29, quad, None)
        for e in range(118, 125):
            slot = e % 4
            do_scatter(e, slot, sync=(e == 124))
            if e + 4 <= 124:
                row_prefetch(e + 4, slot)
            if e + 2 <= 124:
                j = (slot + 2) % 4
                drain(j)
                col_prefetch(e + 2, j)
                gather_fire(j)
        drain(1)
        drain(2)
        drain(3)

        plsc.subcore_barrier()
        obase = cid * NPAD + sid * STRIPE
        pltpu.sync_copy(acc_sh.at[pl.ds(sid * STRIPE, STRIPE)],
                        out_hbm.at[pl.ds(obase, STRIPE)])


def _nt(a, b):
    return lax.dot_general(a, b, (((1,), (1,)), ((), ())),
                           preferred_element_type=jnp.float32)


def _full(shape):
    return pl.BlockSpec(shape, lambda *_: tuple(0 for _ in shape))


def _k0_body(degp_ref, dinv_ref):
    deg = jnp.sum(degp_ref[...], axis=0, keepdims=True) + 2.0
    dlane = lax.rsqrt(deg)
    ones = jnp.ones((1, HID), jnp.float32)
    dinv_ref[...] = lax.dot_general(dlane, ones, (((0,), (0,)), ((), ())),
                                    preferred_element_type=jnp.float32)


_k0 = pl.pallas_call(
    _k0_body,
    grid=(GR,),
    in_specs=[pl.BlockSpec((NW, RB), lambda i: (0, i))],
    out_specs=pl.BlockSpec((RB, HID), lambda i: (i, 0)),
    out_shape=jax.ShapeDtypeStruct((NPAD, HID), jnp.float32),
)


def _ka_body(x_ref, dinv_ref, lin_w_ref, lin_b_ref, c1_w_ref,
             h_ref, z1_ref):
    dinv = dinv_ref[...]
    hs = _nt(x_ref[...], lin_w_ref[...]) + lin_b_ref[...]
    xw = _nt(hs, c1_w_ref[...])
    h_ref[...] = hs
    z1_ref[...] = dinv * xw


def _make_ka(s):
    return pl.pallas_call(
        _ka_body,
        grid=(GR,),
        in_specs=[
            pl.BlockSpec((RB, HID), lambda i: (i, s)),
            pl.BlockSpec((RB, HID), lambda i: (i, 0)),
            _full((HID, HID)),
            _full((1, HID)),
            _full((HID, HID)),
        ],
        out_specs=[
            pl.BlockSpec((RB, HID), lambda i: (i, 0)),
            pl.BlockSpec((RB, HID), lambda i: (i, 0)),
        ],
        out_shape=[
            jax.ShapeDtypeStruct((NPAD, HID), jnp.float32),
            jax.ShapeDtypeStruct((NPAD, HID), jnp.float32),
        ],
    )


_kas = [_make_ka(s) for s in range(3)]


def _kb_body(dinv_ref, h_ref, z1_ref, p_ref, c1_b_ref, c2_w_ref,
             out1_ref, z2_ref):
    dinv = dinv_ref[...]
    agg = p_ref[0] + p_ref[1]
    conv1 = dinv * agg + 2.0 * dinv * z1_ref[...] + c1_b_ref[...]
    o1 = jax.nn.relu(conv1) + h_ref[...]
    out1_ref[...] = o1
    z2_ref[...] = dinv * _nt(o1, c2_w_ref[...])


_kb = pl.pallas_call(
    _kb_body,
    grid=(GR,),
    in_specs=[
        pl.BlockSpec((RB, HID), lambda i: (i, 0)),
        pl.BlockSpec((RB, HID), lambda i: (i, 0)),
        pl.BlockSpec((RB, HID), lambda i: (i, 0)),
        pl.BlockSpec((NC, RB, HID), lambda i: (0, i, 0)),
        _full((1, HID)),
        _full((HID, HID)),
    ],
    out_specs=[
        pl.BlockSpec((RB, HID), lambda i: (i, 0)),
        pl.BlockSpec((RB, HID), lambda i: (i, 0)),
    ],
    out_shape=[
        jax.ShapeDtypeStruct((NPAD, HID), jnp.float32),
        jax.ShapeDtypeStruct((NPAD, HID), jnp.float32),
    ],
)


def _kc_body(dinv_ref, out1_ref, z2_ref, p_ref, c2_b_ref, batch_ref,
             seq_ref, pooled, cnt):
    i = pl.program_id(0)
    dinv = dinv_ref[...]
    agg = p_ref[0] + p_ref[1]
    o2 = (dinv * agg + 2.0 * dinv * z2_ref[...] + c2_b_ref[...]
          + out1_ref[...])
    bt = batch_ref[...]
    ohT = (jnp.broadcast_to(bt, (B, RB))
           == lax.broadcasted_iota(jnp.int32, (B, RB), 0)).astype(jnp.float32)

    @pl.when(i == 0)
    def _():
        pooled[...] = jnp.zeros_like(pooled)
        cnt[...] = jnp.zeros_like(cnt)

    pooled[...] += lax.dot_general(ohT, o2, (((1,), (0,)), ((), ())),
                                   preferred_element_type=jnp.float32)
    cnt[...] += jnp.sum(ohT, axis=1, keepdims=True)

    @pl.when(i == GR - 1)
    def _():
        seq_ref[...] = pooled[...] / jnp.maximum(cnt[...], 1.0)


_kc = pl.pallas_call(
    _kc_body,
    grid=(GR,),
    in_specs=[
        pl.BlockSpec((RB, HID), lambda i: (i, 0)),
        pl.BlockSpec((RB, HID), lambda i: (i, 0)),
        pl.BlockSpec((RB, HID), lambda i: (i, 0)),
        pl.BlockSpec((NC, RB, HID), lambda i: (0, i, 0)),
        _full((1, HID)),
        pl.BlockSpec((1, RB), lambda i: (0, i)),
    ],
    out_specs=_full((B, HID)),
    out_shape=jax.ShapeDtypeStruct((B, HID), jnp.float32),
    scratch_shapes=[
        pltpu.VMEM((B, HID), jnp.float32),
        pltpu.VMEM((B, 1), jnp.float32),
    ],
)


def _kd_body(s0_ref, s1_ref, s2_ref, cov_ref, w_ih_ref, w_hh_ref,
             b_ih_ref, b_hh_ref, w0_w_ref, w0_b_ref, aw1_ref, aw2_ref,
             ab_ref, l1_w_ref, l1_b_ref, l2a_ref, l2b_ref, l2_b_ref,
             h0_ref, c0_ref, attn_ref, out_ref):
    h = h0_ref[...]
    c = c0_ref[...]
    hs = []
    for t in range(3):
        xt = [s0_ref, s1_ref, s2_ref][t][...]
        g = (_nt(xt, w_ih_ref[...]) + b_ih_ref[...]
             + _nt(h, w_hh_ref[...]) + b_hh_ref[...])
        ii = jax.nn.sigmoid(g[:, 0 * HID:1 * HID])
        ff = jax.nn.sigmoid(g[:, 1 * HID:2 * HID])
        gg = jnp.tanh(g[:, 2 * HID:3 * HID])
        oo = jax.nn.sigmoid(g[:, 3 * HID:4 * HID])
        c = ff * c + ii * gg
        h = oo * jnp.tanh(c)
        hs.append(h)
    hT = hs[-1]
    aw1 = aw1_ref[...]
    aw2 = aw2_ref[...]
    ab = ab_ref[0, 0]
    rs, ws = [], []
    for t in range(3):
        rt = jnp.tanh(_nt(hs[t], w0_w_ref[...]) + w0_b_ref[...])
        wt = (jnp.sum(rt * aw1, axis=1, keepdims=True)
              + jnp.sum(hT * aw2, axis=1, keepdims=True) + ab)
        rs.append(rt)
        ws.append(wt)
    m = jnp.maximum(ws[0], jnp.maximum(ws[1], ws[2]))
    es = [jnp.exp(w - m) for w in ws]
    tot = es[0] + es[1] + es[2]
    als = [e / tot for e in es]
    feat = als[0] * rs[0] + als[1] * rs[1] + als[2] * rs[2]
    l1o = jax.nn.relu(_nt(feat, l1_w_ref[...]) + l1_b_ref[...])
    out = (_nt(l1o, l2a_ref[...]) + _nt(cov_ref[...], l2b_ref[...])
           + l2_b_ref[...])
    attn_ref[...] = jnp.concatenate(
        [als[0], als[1], als[2], jnp.zeros((B, HID - 3), jnp.float32)], axis=1)
    out_ref[...] = jnp.concatenate(
        [out, jnp.zeros((B, HID - 2), jnp.float32)], axis=1)


_kd = pl.pallas_call(
    _kd_body,
    out_shape=[
        jax.ShapeDtypeStruct((B, HID), jnp.float32),
        jax.ShapeDtypeStruct((B, HID), jnp.float32),
    ],
)


def kernel(x, edge_index, cov, batch, lin_w, lin_b, c1_w, c1_b, c2_w, c2_b,
           w_ih, w_hh, b_ih, b_hh, w0_w, w0_b, attn_w, attn_b,
           l1_w, l1_b, l2_w, l2_b, h0, c0):
    f32 = jnp.float32
    x_pad = jnp.pad(x, ((0, NPAD - N), (0, 0)))
    batch_pad = jnp.pad(batch, (0, NPAD - N),
                        constant_values=B).reshape(1, NPAD)
    row = edge_index[0]
    col = edge_index[1]

    degp = _deg_call(col)
    dinv_b = _k0(degp)
    lin_b2 = lin_b.reshape(1, HID)
    c1_b2 = c1_b.reshape(1, HID)
    c2_b2 = c2_b.reshape(1, HID)
    seqs = []
    for s in range(3):
        h_s, z1_s = _kas[s](x_pad, dinv_b, lin_w, lin_b2, c1_w)
        p1 = _scatter_call(row, col, z1_s).reshape(NC, NPAD, HID)
        out1_s, z2_s = _kb(dinv_b, h_s, z1_s, p1, c1_b2, c2_w)
        p2 = _scatter_call(row, col, z2_s).reshape(NC, NPAD, HID)
        seqs.append(_kc(dinv_b, out1_s, z2_s, p2, c2_b2, batch_pad))

    attn_p, out_p = _kd(
        seqs[0], seqs[1], seqs[2], cov.astype(f32), w_ih, w_hh,
        b_ih.reshape(1, 4 * HID), b_hh.reshape(1, 4 * HID),
        w0_w, w0_b.reshape(1, HID),
        attn_w[:, :HID], attn_w[:, HID:], attn_b.reshape(1, 1),
        l1_w, l1_b.reshape(1, 8),
        l2_w[:, :8], l2_w[:, 8:], l2_b.reshape(1, 2),
        h0[0], c0[0])
    return (attn_p[:, :3], out_p[:, :2])
